# Initial kernel scaffold; baseline (speedup 1.0000x reference)
import jax
import jax.numpy as jnp
from jax.experimental import pallas as pl


def kernel(x, edge_index, edge_attr, graph_ids, gcn1_W, gcn1_b, gcnb1_W, gcnb1_b, gcnb2_W, gcnb2_b, gcnb2_proj, ecc1_K, ecc1_root, ecc1_b, eccb1_K, eccb1_root, eccb1_b, eccb2_K, eccb2_root, eccb2_b, eccb2_proj, d1_W, d1_b, d2_W, d2_b, d3_W, d3_b):
    raise NotImplementedError("write your pallas kernel here")


# speedup vs baseline: 1.9123x; 1.9123x over previous
import functools

import jax
import jax.numpy as jnp
from jax import lax
from jax.experimental import pallas as pl
from jax.experimental.pallas import tpu as pltpu
from jax.experimental.pallas import tpu_sc as plsc

N = 10000
E = 160000
F = 128
DE = 4
NN = 64
NG = 8
NOUT = 10

NPAD = 10112
NTILE = 16
NCORE = 2
NW = NCORE * NTILE
EP = 163840
EW = EP // NW
RPT = NPAD // NTILE
RB = 632
GR = NPAD // RB
DCH = 64


def _sc_mesh():
    return plsc.VectorSubcoreMesh(core_axis_name="c", subcore_axis_name="s")


def _deg_pass(dstp, cones, zrows16):
    NIT = EW // DCH

    @functools.partial(
        pl.kernel,
        out_type=jax.ShapeDtypeStruct((NCORE, NPAD, 128), jnp.float32),
        mesh=_sc_mesh(),
        scratch_types=[
            pltpu.VMEM_SHARED((NPAD, 128), jnp.float32),
            pltpu.VMEM((DCH,), jnp.int32),
            pltpu.VMEM((DCH, 128), jnp.float32),
        ],
    )
    def k(dst_hbm, cones_hbm, z_hbm, out_hbm, acc, didx, ones_v):
        c = lax.axis_index("c")
        s = lax.axis_index("s")
        wid = c * NTILE + s
        pltpu.sync_copy(cones_hbm, ones_v)
        pltpu.sync_copy(z_hbm, acc.at[pl.ds(s * RPT, RPT)])
        plsc.subcore_barrier()

        def body(it, carry):
            base = wid * EW + it * DCH
            pltpu.sync_copy(dst_hbm.at[pl.ds(base, DCH)], didx)
            pltpu.sync_copy(ones_v, acc.at[didx], add=True)
            return carry

        lax.fori_loop(0, NIT, body, 0)
        plsc.subcore_barrier()
        pltpu.sync_copy(acc.at[pl.ds(s * RPT, RPT)],
                        out_hbm.at[c, pl.ds(s * RPT, RPT)])

    return k(dstp, cones, zrows16)


def _prop_pass(gtab, htab, eap, srcp, dstp, zrows, O):
    HWID = DE * O
    CH = 64 if O <= 64 else 32
    NIT = EW // CH

    @functools.partial(
        pl.kernel,
        out_type=(jax.ShapeDtypeStruct((NCORE, NPAD, 128), jnp.float32),
                  jax.ShapeDtypeStruct((NCORE, NPAD, 128), jnp.float32)),
        mesh=_sc_mesh(),
        scratch_types=[
            pltpu.VMEM_SHARED((NPAD, 128), jnp.float32),
            pltpu.VMEM((CH,), jnp.int32),
            pltpu.VMEM((CH,), jnp.int32),
            pltpu.VMEM((CH, 128), jnp.float32),
            pltpu.VMEM((CH, HWID), jnp.float32),
            pltpu.VMEM((CH, 16), jnp.float32),
            pltpu.SemaphoreType.DMA,
        ],
    )
    def k(g_hbm, h_hbm, ea_hbm, src_hbm, dst_hbm, z_hbm,
          pg_hbm, pe_hbm, acc, sidx, didx, grow, hrow, eav, sem):
        c = lax.axis_index("c")
        s = lax.axis_index("s")
        wid = c * NTILE + s

        def zero_acc():
            pltpu.sync_copy(z_hbm, acc.at[pl.ds(s * RPT, RPT)])

        zero_acc()
        plsc.subcore_barrier()

        def gcn_it(it, carry):
            base = wid * EW + it * CH
            pltpu.sync_copy(src_hbm.at[pl.ds(base, CH)], sidx)
            pltpu.sync_copy(dst_hbm.at[pl.ds(base, CH)], didx)
            pltpu.async_copy(g_hbm.at[sidx], grow, sem).wait()
            pltpu.sync_copy(grow, acc.at[didx], add=True)
            return carry

        lax.fori_loop(0, NIT, gcn_it, 0)
        plsc.subcore_barrier()
        pltpu.sync_copy(acc.at[pl.ds(s * RPT, RPT)],
                        pg_hbm.at[c, pl.ds(s * RPT, RPT)])

        zero_acc()
        if O < 128:
            pltpu.sync_copy(z_hbm.at[pl.ds(0, CH)], grow)
        plsc.subcore_barrier()

        def ecc_it(it, carry):
            base = wid * EW + it * CH
            pltpu.sync_copy(src_hbm.at[pl.ds(base, CH)], sidx)
            pltpu.sync_copy(dst_hbm.at[pl.ds(base, CH)], didx)
            pltpu.sync_copy(ea_hbm.at[pl.ds(base, CH)], eav)
            pltpu.async_copy(h_hbm.at[sidx], hrow, sem).wait()

            def edge(i, ecarry):
                ev = eav[i, :]
                e0 = ev[0]
                e1 = ev[1]
                e2 = ev[2]
                e3 = ev[3]
                for j in range(O // 16):
                    v = (e0 * hrow[i, pl.ds(j * 16, 16)]
                         + e1 * hrow[i, pl.ds(O + j * 16, 16)]
                         + e2 * hrow[i, pl.ds(2 * O + j * 16, 16)]
                         + e3 * hrow[i, pl.ds(3 * O + j * 16, 16)])
                    grow[i, pl.ds(j * 16, 16)] = v
                return ecarry

            lax.fori_loop(0, CH, edge, 0)
            pltpu.sync_copy(grow, acc.at[didx], add=True)
            return carry

        lax.fori_loop(0, NIT, ecc_it, 0)
        plsc.subcore_barrier()
        pltpu.sync_copy(acc.at[pl.ds(s * RPT, RPT)],
                        pe_hbm.at[c, pl.ds(s * RPT, RPT)])

    return k(gtab, htab, eap, srcp, dstp, zrows)


def _full(shape):
    return pl.BlockSpec(shape, lambda i: tuple(0 for _ in shape))


def _tc1(degp, xp, w1, k1f, r1w):
    def body(degp_ref, x_ref, w1_ref, k1_ref, r1_ref,
             dis_ref, g1_ref, h1_ref, rr1_ref):
        d = degp_ref[0] + degp_ref[1]
        dis = lax.rsqrt(jnp.maximum(d[:, 0:1], 1.0))
        dis_ref[...] = dis
        xb = x_ref[...]
        g1_ref[...] = dis * jnp.dot(xb, w1_ref[...],
                                    preferred_element_type=jnp.float32)
        h1_ref[...] = jnp.dot(xb, k1_ref[...],
                              preferred_element_type=jnp.float32)
        rr1_ref[...] = jnp.dot(xb, r1_ref[...],
                               preferred_element_type=jnp.float32)

    return pl.pallas_call(
        body,
        grid=(GR,),
        in_specs=[
            pl.BlockSpec((NCORE, RB, 128), lambda i: (0, i, 0)),
            pl.BlockSpec((RB, F), lambda i: (i, 0)),
            _full((F, 128)),
            _full((F, DE * NN)),
            _full((F, NN)),
        ],
        out_specs=[
            pl.BlockSpec((RB, 1), lambda i: (i, 0)),
            pl.BlockSpec((RB, 128), lambda i: (i, 0)),
            pl.BlockSpec((RB, DE * NN), lambda i: (i, 0)),
            pl.BlockSpec((RB, NN), lambda i: (i, 0)),
        ],
        out_shape=[
            jax.ShapeDtypeStruct((NPAD, 1), jnp.float32),
            jax.ShapeDtypeStruct((NPAD, 128), jnp.float32),
            jax.ShapeDtypeStruct((NPAD, DE * NN), jnp.float32),
            jax.ShapeDtypeStruct((NPAD, NN), jnp.float32),
        ],
    )(degp, xp, w1, k1f, r1w)


def _tc2(p1g, p1e, rr1, dis, b1, be1, wb1, kb1f, rb1w):
    def body(pg_ref, pe_ref, rr1_ref, dis_ref, b1_ref, be1_ref,
             wb1_ref, kb1_ref, rb1_ref,
             o1_ref, o2_ref, g2_ref, h2_ref, rr2_ref):
        dis = dis_ref[...]
        o1 = jnp.maximum((pg_ref[0, :, :NN] + pg_ref[1, :, :NN]) * dis
                         + b1_ref[...], 0.0)
        o2 = jnp.maximum(pe_ref[0, :, :NN] + pe_ref[1, :, :NN]
                         + rr1_ref[...] + be1_ref[...], 0.0)
        o1_ref[...] = o1
        o2_ref[...] = o2
        g2_ref[...] = dis * jnp.dot(o1, wb1_ref[...],
                                    preferred_element_type=jnp.float32)
        h2_ref[...] = jnp.dot(o2, kb1_ref[...],
                              preferred_element_type=jnp.float32)
        rr2_ref[...] = jnp.dot(o2, rb1_ref[...],
                               preferred_element_type=jnp.float32)

    return pl.pallas_call(
        body,
        grid=(GR,),
        in_specs=[
            pl.BlockSpec((NCORE, RB, 128), lambda i: (0, i, 0)),
            pl.BlockSpec((NCORE, RB, 128), lambda i: (0, i, 0)),
            pl.BlockSpec((RB, NN), lambda i: (i, 0)),
            pl.BlockSpec((RB, 1), lambda i: (i, 0)),
            _full((1, NN)),
            _full((1, NN)),
            _full((NN, 128)),
            _full((NN, DE * NN)),
            _full((NN, NN)),
        ],
        out_specs=[
            pl.BlockSpec((RB, NN), lambda i: (i, 0)),
            pl.BlockSpec((RB, NN), lambda i: (i, 0)),
            pl.BlockSpec((RB, 128), lambda i: (i, 0)),
            pl.BlockSpec((RB, DE * NN), lambda i: (i, 0)),
            pl.BlockSpec((RB, NN), lambda i: (i, 0)),
        ],
        out_shape=[
            jax.ShapeDtypeStruct((NPAD, NN), jnp.float32),
            jax.ShapeDtypeStruct((NPAD, NN), jnp.float32),
            jax.ShapeDtypeStruct((NPAD, 128), jnp.float32),
            jax.ShapeDtypeStruct((NPAD, DE * NN), jnp.float32),
            jax.ShapeDtypeStruct((NPAD, NN), jnp.float32),
        ],
    )(p1g, p1e, rr1, dis, b1, be1, wb1, kb1f, rb1w)


def _tc3(p2g, p2e, o1, o2, rr2, dis, bb1, beb1, wb2, kb2f, rb2w, proj1, proj2):
    NO = 2 * NN

    def body(pg_ref, pe_ref, o1_ref, o2_ref, rr2_ref, dis_ref,
             bb1_ref, beb1_ref, wb2_ref, kb2_ref, rb2_ref, pj1_ref, pj2_ref,
             g3_ref, h3_ref, rr3_ref, pr1_ref, pr2_ref):
        dis = dis_ref[...]
        o1b = jnp.maximum((pg_ref[0, :, :NN] + pg_ref[1, :, :NN]) * dis
                          + bb1_ref[...] + o1_ref[...], 0.0)
        o2b = jnp.maximum(pe_ref[0, :, :NN] + pe_ref[1, :, :NN]
                          + rr2_ref[...] + beb1_ref[...] + o2_ref[...], 0.0)
        g3_ref[...] = dis * jnp.dot(o1b, wb2_ref[...],
                                    preferred_element_type=jnp.float32)
        h3_ref[...] = jnp.dot(o2b, kb2_ref[...],
                              preferred_element_type=jnp.float32)
        rr3_ref[...] = jnp.dot(o2b, rb2_ref[...],
                               preferred_element_type=jnp.float32)
        pr1_ref[...] = jnp.dot(o1b, pj1_ref[...],
                               preferred_element_type=jnp.float32)
        pr2_ref[...] = jnp.dot(o2b, pj2_ref[...],
                               preferred_element_type=jnp.float32)

    return pl.pallas_call(
        body,
        grid=(GR,),
        in_specs=[
            pl.BlockSpec((NCORE, RB, 128), lambda i: (0, i, 0)),
            pl.BlockSpec((NCORE, RB, 128), lambda i: (0, i, 0)),
            pl.BlockSpec((RB, NN), lambda i: (i, 0)),
            pl.BlockSpec((RB, NN), lambda i: (i, 0)),
            pl.BlockSpec((RB, NN), lambda i: (i, 0)),
            pl.BlockSpec((RB, 1), lambda i: (i, 0)),
            _full((1, NN)),
            _full((1, NN)),
            _full((NN, NO)),
            _full((NN, DE * NO)),
            _full((NN, NO)),
            _full((NN, NO)),
            _full((NN, NO)),
        ],
        out_specs=[
            pl.BlockSpec((RB, NO), lambda i: (i, 0)),
            pl.BlockSpec((RB, DE * NO), lambda i: (i, 0)),
            pl.BlockSpec((RB, NO), lambda i: (i, 0)),
            pl.BlockSpec((RB, NO), lambda i: (i, 0)),
            pl.BlockSpec((RB, NO), lambda i: (i, 0)),
        ],
        out_shape=[
            jax.ShapeDtypeStruct((NPAD, NO), jnp.float32),
            jax.ShapeDtypeStruct((NPAD, DE * NO), jnp.float32),
            jax.ShapeDtypeStruct((NPAD, NO), jnp.float32),
            jax.ShapeDtypeStruct((NPAD, NO), jnp.float32),
            jax.ShapeDtypeStruct((NPAD, NO), jnp.float32),
        ],
    )(p2g, p2e, o1, o2, rr2, dis, bb1, beb1, wb2, kb2f, rb2w, proj1, proj2)


def _tc4(p3g, p3e, rr3, pr1, pr2, dis, bb2, beb2, gids,
         d1w, d1b, d2w, d2b, d3wp, d3bp):
    NO = 2 * NN

    def body(pg_ref, pe_ref, rr3_ref, pr1_ref, pr2_ref, dis_ref,
             bb2_ref, beb2_ref, gid_ref, d1w_ref, d1b_ref, d2w_ref, d2b_ref,
             d3w_ref, d3b_ref, out_ref, acc_ref):
        i = pl.program_id(0)

        @pl.when(i == 0)
        def _():
            acc_ref[...] = jnp.zeros_like(acc_ref)

        dis = dis_ref[...]
        o1c = jnp.maximum((pg_ref[0] + pg_ref[1]) * dis + bb2_ref[...]
                          + pr1_ref[...], 0.0)
        o2c = jnp.maximum(pe_ref[0] + pe_ref[1] + rr3_ref[...] + beb2_ref[...]
                          + pr2_ref[...], 0.0)
        seg = (gid_ref[...] == lax.broadcasted_iota(jnp.int32, (RB, NG), 1)
               ).astype(jnp.float32)
        dn = (((0,), (0,)), ((), ()))
        acc_ref[:, 0:NO] += lax.dot_general(
            seg, o1c, dn, preferred_element_type=jnp.float32)
        acc_ref[:, NO:2 * NO] += lax.dot_general(
            seg, o2c, dn, preferred_element_type=jnp.float32)

        @pl.when(i == GR - 1)
        def _():
            p = acc_ref[...]
            h1 = jnp.maximum(
                jnp.dot(p, d1w_ref[...], preferred_element_type=jnp.float32)
                + d1b_ref[...], 0.0)
            h2 = jnp.maximum(
                jnp.dot(h1, d2w_ref[...], preferred_element_type=jnp.float32)
                + d2b_ref[...], 0.0)
            lg = jnp.dot(h2, d3w_ref[...],
                         preferred_element_type=jnp.float32) + d3b_ref[...]
            mx = jnp.max(lg, axis=1, keepdims=True)
            ex = jnp.exp(lg - mx)
            out_ref[...] = ex / jnp.sum(ex, axis=1, keepdims=True)

    return pl.pallas_call(
        body,
        grid=(GR,),
        in_specs=[
            pl.BlockSpec((NCORE, RB, NO), lambda i: (0, i, 0)),
            pl.BlockSpec((NCORE, RB, NO), lambda i: (0, i, 0)),
            pl.BlockSpec((RB, NO), lambda i: (i, 0)),
            pl.BlockSpec((RB, NO), lambda i: (i, 0)),
            pl.BlockSpec((RB, NO), lambda i: (i, 0)),
            pl.BlockSpec((RB, 1), lambda i: (i, 0)),
            _full((1, NO)),
            _full((1, NO)),
            pl.BlockSpec((RB, 1), lambda i: (i, 0)),
            _full((2 * NO, NN)),
            _full((1, NN)),
            _full((NN, NN // 2)),
            _full((1, NN // 2)),
            _full((NN // 2, 128)),
            _full((1, 128)),
        ],
        out_specs=pl.BlockSpec((NG, 128), lambda i: (0, 0)),
        out_shape=jax.ShapeDtypeStruct((NG, 128), jnp.float32),
        scratch_shapes=[pltpu.VMEM((NG, 2 * NO), jnp.float32)],
    )(p3g, p3e, rr3, pr1, pr2, dis, bb2, beb2, gids,
      d1w, d1b, d2w, d2b, d3wp, d3bp)


def kernel(x, edge_index, edge_attr, graph_ids,
           gcn1_W, gcn1_b, gcnb1_W, gcnb1_b, gcnb2_W, gcnb2_b, gcnb2_proj,
           ecc1_K, ecc1_root, ecc1_b, eccb1_K, eccb1_root, eccb1_b,
           eccb2_K, eccb2_root, eccb2_b, eccb2_proj,
           d1_W, d1_b, d2_W, d2_b, d3_W, d3_b):
    padE = EP - E
    srcp = jnp.concatenate([edge_index[0], jnp.zeros((padE,), jnp.int32)])
    dstp = jnp.concatenate([edge_index[1], jnp.full((padE,), N, jnp.int32)])
    eap = jnp.pad(edge_attr, ((0, padE), (0, 16 - DE)))
    xp = jnp.pad(x, ((0, NPAD - N), (0, 0)))
    gidsp = jnp.concatenate(
        [graph_ids, jnp.full((NPAD - N,), NG, jnp.int32)]).reshape(NPAD, 1)

    k1f = jnp.transpose(ecc1_K, (1, 0, 2)).reshape(F, DE * NN)
    kb1f = jnp.transpose(eccb1_K, (1, 0, 2)).reshape(NN, DE * NN)
    kb2f = jnp.transpose(eccb2_K, (1, 0, 2)).reshape(NN, DE * 2 * NN)

    cones = jnp.concatenate(
        [jnp.ones((DCH, 1), jnp.float32), jnp.zeros((DCH, 127), jnp.float32)],
        axis=1)
    z128 = jnp.zeros((RPT, 128), jnp.float32)
    w1p = jnp.pad(gcn1_W, ((0, 0), (0, 128 - NN)))
    wb1p = jnp.pad(gcnb1_W, ((0, 0), (0, 128 - NN)))

    b1 = gcn1_b.reshape(1, NN)
    bb1 = gcnb1_b.reshape(1, NN)
    bb2 = gcnb2_b.reshape(1, 2 * NN)
    be1 = ecc1_b.reshape(1, NN)
    beb1 = eccb1_b.reshape(1, NN)
    beb2 = eccb2_b.reshape(1, 2 * NN)
    d1b = d1_b.reshape(1, NN)
    d2b = d2_b.reshape(1, NN // 2)
    d3wp = jnp.pad(d3_W, ((0, 0), (0, 128 - NOUT)))
    d3bp = jnp.concatenate(
        [d3_b, jnp.full((128 - NOUT,), -1e30, jnp.float32)]).reshape(1, 128)

    degp = _deg_pass(dstp, cones, z128)
    dis, g1, h1, rr1 = _tc1(degp, xp, w1p, k1f, ecc1_root)
    p1g, p1e = _prop_pass(g1, h1, eap, srcp, dstp, z128, NN)
    o1, o2, g2, h2, rr2 = _tc2(p1g, p1e, rr1, dis, b1, be1,
                               wb1p, kb1f, eccb1_root)
    p2g, p2e = _prop_pass(g2, h2, eap, srcp, dstp, z128, NN)
    g3, h3, rr3, pr1, pr2 = _tc3(p2g, p2e, o1, o2, rr2, dis, bb1, beb1,
                                 gcnb2_W, kb2f, eccb2_root,
                                 gcnb2_proj, eccb2_proj)
    p3g, p3e = _prop_pass(g3, h3, eap, srcp, dstp, z128, 2 * NN)
    outp = _tc4(p3g, p3e, rr3, pr1, pr2, dis, bb2, beb2, gidsp,
                d1_W, d1b, d2_W, d2b, d3wp, d3bp)
    return outp[:, :NOUT]


# speedup vs baseline: 2.0443x; 1.0690x over previous
import functools

import jax
import jax.numpy as jnp
from jax import lax
from jax.experimental import pallas as pl
from jax.experimental.pallas import tpu as pltpu
from jax.experimental.pallas import tpu_sc as plsc

N = 10000
E = 160000
F = 128
DE = 4
NN = 64
NG = 8
NOUT = 10

NPAD = 10112
NTILE = 16
NCORE = 2
NW = NCORE * NTILE
EP = 163840
EW = EP // NW
RPT = NPAD // NTILE
RB = 632
GR = NPAD // RB
DCH = 64


def _sc_mesh():
    return plsc.VectorSubcoreMesh(core_axis_name="c", subcore_axis_name="s")


def _deg_pass(dstp, cones, zrows16):
    NIT = EW // DCH

    @functools.partial(
        pl.kernel,
        out_type=jax.ShapeDtypeStruct((NCORE, NPAD, 128), jnp.float32),
        mesh=_sc_mesh(),
        scratch_types=[
            pltpu.VMEM_SHARED((NPAD, 128), jnp.float32),
            pltpu.VMEM((DCH,), jnp.int32),
            pltpu.VMEM((DCH, 128), jnp.float32),
        ],
    )
    def k(dst_hbm, cones_hbm, z_hbm, out_hbm, acc, didx, ones_v):
        c = lax.axis_index("c")
        s = lax.axis_index("s")
        wid = c * NTILE + s
        pltpu.sync_copy(cones_hbm, ones_v)
        pltpu.sync_copy(z_hbm, acc.at[pl.ds(s * RPT, RPT)])
        plsc.subcore_barrier()

        def body(it, carry):
            base = wid * EW + it * DCH
            pltpu.sync_copy(dst_hbm.at[pl.ds(base, DCH)], didx)
            pltpu.sync_copy(ones_v, acc.at[didx], add=True)
            return carry

        lax.fori_loop(0, NIT, body, 0)
        plsc.subcore_barrier()
        pltpu.sync_copy(acc.at[pl.ds(s * RPT, RPT)],
                        out_hbm.at[c, pl.ds(s * RPT, RPT)])

    return k(dstp, cones, zrows16)


def _prop_pass(gtab, htab, eap, srcp, dstp, zrows, O):
    HWID = DE * O
    CH = 64
    NIT = EW // CH

    @functools.partial(
        pl.kernel,
        out_type=(jax.ShapeDtypeStruct((NCORE, NPAD, 128), jnp.float32),
                  jax.ShapeDtypeStruct((NCORE, NPAD, 128), jnp.float32)),
        mesh=_sc_mesh(),
        scratch_types=[
            pltpu.VMEM_SHARED((NPAD, 128), jnp.float32),
            pltpu.VMEM((CH,), jnp.int32),
            pltpu.VMEM((CH,), jnp.int32),
            pltpu.VMEM((CH, 128), jnp.float32),
            pltpu.VMEM((CH, HWID), jnp.float32),
            pltpu.VMEM((CH, 16), jnp.float32),
            pltpu.SemaphoreType.DMA,
        ],
    )
    def k(g_hbm, h_hbm, ea_hbm, src_hbm, dst_hbm, z_hbm,
          pg_hbm, pe_hbm, acc, sidx, didx, grow, hrow, eav, sem):
        c = lax.axis_index("c")
        s = lax.axis_index("s")
        wid = c * NTILE + s

        def zero_acc():
            pltpu.sync_copy(z_hbm, acc.at[pl.ds(s * RPT, RPT)])

        zero_acc()
        plsc.subcore_barrier()

        def gcn_it(it, carry):
            base = wid * EW + it * CH
            pltpu.sync_copy(src_hbm.at[pl.ds(base, CH)], sidx)
            pltpu.sync_copy(dst_hbm.at[pl.ds(base, CH)], didx)
            pltpu.async_copy(g_hbm.at[sidx], grow, sem).wait()
            pltpu.sync_copy(grow, acc.at[didx], add=True)
            return carry

        lax.fori_loop(0, NIT, gcn_it, 0)
        plsc.subcore_barrier()
        pltpu.sync_copy(acc.at[pl.ds(s * RPT, RPT)],
                        pg_hbm.at[c, pl.ds(s * RPT, RPT)])

        zero_acc()
        if O < 128:
            pltpu.sync_copy(z_hbm.at[pl.ds(0, CH)], grow)
        plsc.subcore_barrier()

        def ecc_it(it, carry):
            base = wid * EW + it * CH
            pltpu.sync_copy(src_hbm.at[pl.ds(base, CH)], sidx)
            pltpu.sync_copy(dst_hbm.at[pl.ds(base, CH)], didx)
            pltpu.sync_copy(ea_hbm.at[pl.ds(base, CH)], eav)
            pltpu.async_copy(h_hbm.at[sidx], hrow, sem).wait()

            def edge(i, ecarry):
                ev = eav[i, :]
                e0 = ev[0]
                e1 = ev[1]
                e2 = ev[2]
                e3 = ev[3]
                for j in range(O // 16):
                    v = (e0 * hrow[i, pl.ds(j * 16, 16)]
                         + e1 * hrow[i, pl.ds(O + j * 16, 16)]
                         + e2 * hrow[i, pl.ds(2 * O + j * 16, 16)]
                         + e3 * hrow[i, pl.ds(3 * O + j * 16, 16)])
                    grow[i, pl.ds(j * 16, 16)] = v
                return ecarry

            lax.fori_loop(0, CH, edge, 0)
            pltpu.sync_copy(grow, acc.at[didx], add=True)
            return carry

        lax.fori_loop(0, NIT, ecc_it, 0)
        plsc.subcore_barrier()
        pltpu.sync_copy(acc.at[pl.ds(s * RPT, RPT)],
                        pe_hbm.at[c, pl.ds(s * RPT, RPT)])

    return k(gtab, htab, eap, srcp, dstp, zrows)


def _full(shape):
    return pl.BlockSpec(shape, lambda i: tuple(0 for _ in shape))


def _tc1(degp, xp, w1, k1f, r1w):
    def body(degp_ref, x_ref, w1_ref, k1_ref, r1_ref,
             dis_ref, g1_ref, h1_ref, rr1_ref):
        d = degp_ref[0] + degp_ref[1]
        dis = lax.rsqrt(jnp.maximum(d[:, 0:1], 1.0))
        dis_ref[...] = dis
        xb = x_ref[...]
        g1_ref[...] = dis * jnp.dot(xb, w1_ref[...],
                                    preferred_element_type=jnp.float32)
        h1_ref[...] = jnp.dot(xb, k1_ref[...],
                              preferred_element_type=jnp.float32)
        rr1_ref[...] = jnp.dot(xb, r1_ref[...],
                               preferred_element_type=jnp.float32)

    return pl.pallas_call(
        body,
        grid=(GR,),
        in_specs=[
            pl.BlockSpec((NCORE, RB, 128), lambda i: (0, i, 0)),
            pl.BlockSpec((RB, F), lambda i: (i, 0)),
            _full((F, 128)),
            _full((F, DE * NN)),
            _full((F, NN)),
        ],
        out_specs=[
            pl.BlockSpec((RB, 1), lambda i: (i, 0)),
            pl.BlockSpec((RB, 128), lambda i: (i, 0)),
            pl.BlockSpec((RB, DE * NN), lambda i: (i, 0)),
            pl.BlockSpec((RB, NN), lambda i: (i, 0)),
        ],
        out_shape=[
            jax.ShapeDtypeStruct((NPAD, 1), jnp.float32),
            jax.ShapeDtypeStruct((NPAD, 128), jnp.float32),
            jax.ShapeDtypeStruct((NPAD, DE * NN), jnp.float32),
            jax.ShapeDtypeStruct((NPAD, NN), jnp.float32),
        ],
    )(degp, xp, w1, k1f, r1w)


def _tc2(p1g, p1e, rr1, dis, b1, be1, wb1, kb1f, rb1w):
    def body(pg_ref, pe_ref, rr1_ref, dis_ref, b1_ref, be1_ref,
             wb1_ref, kb1_ref, rb1_ref,
             o1_ref, o2_ref, g2_ref, h2_ref, rr2_ref):
        dis = dis_ref[...]
        o1 = jnp.maximum((pg_ref[0, :, :NN] + pg_ref[1, :, :NN]) * dis
                         + b1_ref[...], 0.0)
        o2 = jnp.maximum(pe_ref[0, :, :NN] + pe_ref[1, :, :NN]
                         + rr1_ref[...] + be1_ref[...], 0.0)
        o1_ref[...] = o1
        o2_ref[...] = o2
        g2_ref[...] = dis * jnp.dot(o1, wb1_ref[...],
                                    preferred_element_type=jnp.float32)
        h2_ref[...] = jnp.dot(o2, kb1_ref[...],
                              preferred_element_type=jnp.float32)
        rr2_ref[...] = jnp.dot(o2, rb1_ref[...],
                               preferred_element_type=jnp.float32)

    return pl.pallas_call(
        body,
        grid=(GR,),
        in_specs=[
            pl.BlockSpec((NCORE, RB, 128), lambda i: (0, i, 0)),
            pl.BlockSpec((NCORE, RB, 128), lambda i: (0, i, 0)),
            pl.BlockSpec((RB, NN), lambda i: (i, 0)),
            pl.BlockSpec((RB, 1), lambda i: (i, 0)),
            _full((1, NN)),
            _full((1, NN)),
            _full((NN, 128)),
            _full((NN, DE * NN)),
            _full((NN, NN)),
        ],
        out_specs=[
            pl.BlockSpec((RB, NN), lambda i: (i, 0)),
            pl.BlockSpec((RB, NN), lambda i: (i, 0)),
            pl.BlockSpec((RB, 128), lambda i: (i, 0)),
            pl.BlockSpec((RB, DE * NN), lambda i: (i, 0)),
            pl.BlockSpec((RB, NN), lambda i: (i, 0)),
        ],
        out_shape=[
            jax.ShapeDtypeStruct((NPAD, NN), jnp.float32),
            jax.ShapeDtypeStruct((NPAD, NN), jnp.float32),
            jax.ShapeDtypeStruct((NPAD, 128), jnp.float32),
            jax.ShapeDtypeStruct((NPAD, DE * NN), jnp.float32),
            jax.ShapeDtypeStruct((NPAD, NN), jnp.float32),
        ],
    )(p1g, p1e, rr1, dis, b1, be1, wb1, kb1f, rb1w)


def _tc3(p2g, p2e, o1, o2, rr2, dis, bb1, beb1, wb2, kb2f, rb2w, proj1, proj2):
    NO = 2 * NN

    def body(pg_ref, pe_ref, o1_ref, o2_ref, rr2_ref, dis_ref,
             bb1_ref, beb1_ref, wb2_ref, kb2_ref, rb2_ref, pj1_ref, pj2_ref,
             g3_ref, h3_ref, rr3_ref, pr1_ref, pr2_ref):
        dis = dis_ref[...]
        o1b = jnp.maximum((pg_ref[0, :, :NN] + pg_ref[1, :, :NN]) * dis
                          + bb1_ref[...] + o1_ref[...], 0.0)
        o2b = jnp.maximum(pe_ref[0, :, :NN] + pe_ref[1, :, :NN]
                          + rr2_ref[...] + beb1_ref[...] + o2_ref[...], 0.0)
        g3_ref[...] = dis * jnp.dot(o1b, wb2_ref[...],
                                    preferred_element_type=jnp.float32)
        h3_ref[...] = jnp.dot(o2b, kb2_ref[...],
                              preferred_element_type=jnp.float32)
        rr3_ref[...] = jnp.dot(o2b, rb2_ref[...],
                               preferred_element_type=jnp.float32)
        pr1_ref[...] = jnp.dot(o1b, pj1_ref[...],
                               preferred_element_type=jnp.float32)
        pr2_ref[...] = jnp.dot(o2b, pj2_ref[...],
                               preferred_element_type=jnp.float32)

    return pl.pallas_call(
        body,
        grid=(GR,),
        in_specs=[
            pl.BlockSpec((NCORE, RB, 128), lambda i: (0, i, 0)),
            pl.BlockSpec((NCORE, RB, 128), lambda i: (0, i, 0)),
            pl.BlockSpec((RB, NN), lambda i: (i, 0)),
            pl.BlockSpec((RB, NN), lambda i: (i, 0)),
            pl.BlockSpec((RB, NN), lambda i: (i, 0)),
            pl.BlockSpec((RB, 1), lambda i: (i, 0)),
            _full((1, NN)),
            _full((1, NN)),
            _full((NN, NO)),
            _full((NN, DE * NO)),
            _full((NN, NO)),
            _full((NN, NO)),
            _full((NN, NO)),
        ],
        out_specs=[
            pl.BlockSpec((RB, NO), lambda i: (i, 0)),
            pl.BlockSpec((RB, DE * NO), lambda i: (i, 0)),
            pl.BlockSpec((RB, NO), lambda i: (i, 0)),
            pl.BlockSpec((RB, NO), lambda i: (i, 0)),
            pl.BlockSpec((RB, NO), lambda i: (i, 0)),
        ],
        out_shape=[
            jax.ShapeDtypeStruct((NPAD, NO), jnp.float32),
            jax.ShapeDtypeStruct((NPAD, DE * NO), jnp.float32),
            jax.ShapeDtypeStruct((NPAD, NO), jnp.float32),
            jax.ShapeDtypeStruct((NPAD, NO), jnp.float32),
            jax.ShapeDtypeStruct((NPAD, NO), jnp.float32),
        ],
    )(p2g, p2e, o1, o2, rr2, dis, bb1, beb1, wb2, kb2f, rb2w, proj1, proj2)


def _tc4(p3g, p3e, rr3, pr1, pr2, dis, bb2, beb2, gids,
         d1w, d1b, d2w, d2b, d3wp, d3bp):
    NO = 2 * NN

    def body(pg_ref, pe_ref, rr3_ref, pr1_ref, pr2_ref, dis_ref,
             bb2_ref, beb2_ref, gid_ref, d1w_ref, d1b_ref, d2w_ref, d2b_ref,
             d3w_ref, d3b_ref, out_ref, acc_ref):
        i = pl.program_id(0)

        @pl.when(i == 0)
        def _():
            acc_ref[...] = jnp.zeros_like(acc_ref)

        dis = dis_ref[...]
        o1c = jnp.maximum((pg_ref[0] + pg_ref[1]) * dis + bb2_ref[...]
                          + pr1_ref[...], 0.0)
        o2c = jnp.maximum(pe_ref[0] + pe_ref[1] + rr3_ref[...] + beb2_ref[...]
                          + pr2_ref[...], 0.0)
        seg = (gid_ref[...] == lax.broadcasted_iota(jnp.int32, (RB, NG), 1)
               ).astype(jnp.float32)
        dn = (((0,), (0,)), ((), ()))
        acc_ref[:, 0:NO] += lax.dot_general(
            seg, o1c, dn, preferred_element_type=jnp.float32)
        acc_ref[:, NO:2 * NO] += lax.dot_general(
            seg, o2c, dn, preferred_element_type=jnp.float32)

        @pl.when(i == GR - 1)
        def _():
            p = acc_ref[...]
            h1 = jnp.maximum(
                jnp.dot(p, d1w_ref[...], preferred_element_type=jnp.float32)
                + d1b_ref[...], 0.0)
            h2 = jnp.maximum(
                jnp.dot(h1, d2w_ref[...], preferred_element_type=jnp.float32)
                + d2b_ref[...], 0.0)
            lg = jnp.dot(h2, d3w_ref[...],
                         preferred_element_type=jnp.float32) + d3b_ref[...]
            mx = jnp.max(lg, axis=1, keepdims=True)
            ex = jnp.exp(lg - mx)
            out_ref[...] = ex / jnp.sum(ex, axis=1, keepdims=True)

    return pl.pallas_call(
        body,
        grid=(GR,),
        in_specs=[
            pl.BlockSpec((NCORE, RB, NO), lambda i: (0, i, 0)),
            pl.BlockSpec((NCORE, RB, NO), lambda i: (0, i, 0)),
            pl.BlockSpec((RB, NO), lambda i: (i, 0)),
            pl.BlockSpec((RB, NO), lambda i: (i, 0)),
            pl.BlockSpec((RB, NO), lambda i: (i, 0)),
            pl.BlockSpec((RB, 1), lambda i: (i, 0)),
            _full((1, NO)),
            _full((1, NO)),
            pl.BlockSpec((RB, 1), lambda i: (i, 0)),
            _full((2 * NO, NN)),
            _full((1, NN)),
            _full((NN, NN // 2)),
            _full((1, NN // 2)),
            _full((NN // 2, 128)),
            _full((1, 128)),
        ],
        out_specs=pl.BlockSpec((NG, 128), lambda i: (0, 0)),
        out_shape=jax.ShapeDtypeStruct((NG, 128), jnp.float32),
        scratch_shapes=[pltpu.VMEM((NG, 2 * NO), jnp.float32)],
    )(p3g, p3e, rr3, pr1, pr2, dis, bb2, beb2, gids,
      d1w, d1b, d2w, d2b, d3wp, d3bp)


def kernel(x, edge_index, edge_attr, graph_ids,
           gcn1_W, gcn1_b, gcnb1_W, gcnb1_b, gcnb2_W, gcnb2_b, gcnb2_proj,
           ecc1_K, ecc1_root, ecc1_b, eccb1_K, eccb1_root, eccb1_b,
           eccb2_K, eccb2_root, eccb2_b, eccb2_proj,
           d1_W, d1_b, d2_W, d2_b, d3_W, d3_b):
    padE = EP - E
    srcp = jnp.concatenate([edge_index[0], jnp.zeros((padE,), jnp.int32)])
    dstp = jnp.concatenate([edge_index[1], jnp.full((padE,), N, jnp.int32)])
    eap = jnp.pad(edge_attr, ((0, padE), (0, 16 - DE)))
    xp = jnp.pad(x, ((0, NPAD - N), (0, 0)))
    gidsp = jnp.concatenate(
        [graph_ids, jnp.full((NPAD - N,), NG, jnp.int32)]).reshape(NPAD, 1)

    k1f = jnp.transpose(ecc1_K, (1, 0, 2)).reshape(F, DE * NN)
    kb1f = jnp.transpose(eccb1_K, (1, 0, 2)).reshape(NN, DE * NN)
    kb2f = jnp.transpose(eccb2_K, (1, 0, 2)).reshape(NN, DE * 2 * NN)

    cones = jnp.concatenate(
        [jnp.ones((DCH, 1), jnp.float32), jnp.zeros((DCH, 127), jnp.float32)],
        axis=1)
    z128 = jnp.zeros((RPT, 128), jnp.float32)
    w1p = jnp.pad(gcn1_W, ((0, 0), (0, 128 - NN)))
    wb1p = jnp.pad(gcnb1_W, ((0, 0), (0, 128 - NN)))

    b1 = gcn1_b.reshape(1, NN)
    bb1 = gcnb1_b.reshape(1, NN)
    bb2 = gcnb2_b.reshape(1, 2 * NN)
    be1 = ecc1_b.reshape(1, NN)
    beb1 = eccb1_b.reshape(1, NN)
    beb2 = eccb2_b.reshape(1, 2 * NN)
    d1b = d1_b.reshape(1, NN)
    d2b = d2_b.reshape(1, NN // 2)
    d3wp = jnp.pad(d3_W, ((0, 0), (0, 128 - NOUT)))
    d3bp = jnp.concatenate(
        [d3_b, jnp.full((128 - NOUT,), -1e30, jnp.float32)]).reshape(1, 128)

    degp = _deg_pass(dstp, cones, z128)
    dis, g1, h1, rr1 = _tc1(degp, xp, w1p, k1f, ecc1_root)
    p1g, p1e = _prop_pass(g1, h1, eap, srcp, dstp, z128, NN)
    o1, o2, g2, h2, rr2 = _tc2(p1g, p1e, rr1, dis, b1, be1,
                               wb1p, kb1f, eccb1_root)
    p2g, p2e = _prop_pass(g2, h2, eap, srcp, dstp, z128, NN)
    g3, h3, rr3, pr1, pr2 = _tc3(p2g, p2e, o1, o2, rr2, dis, bb1, beb1,
                                 gcnb2_W, kb2f, eccb2_root,
                                 gcnb2_proj, eccb2_proj)
    p3g, p3e = _prop_pass(g3, h3, eap, srcp, dstp, z128, 2 * NN)
    outp = _tc4(p3g, p3e, rr3, pr1, pr2, dis, bb2, beb2, gidsp,
                d1_W, d1b, d2_W, d2b, d3wp, d3bp)
    return outp[:, :NOUT]


# speedup vs baseline: 2.4671x; 1.2068x over previous
import functools

import jax
import jax.numpy as jnp
from jax import lax
from jax.experimental import pallas as pl
from jax.experimental.pallas import tpu as pltpu
from jax.experimental.pallas import tpu_sc as plsc

N = 10000
E = 160000
F = 128
DE = 4
NN = 64
NG = 8
NOUT = 10

NPAD = 10112
NTILE = 16
NCORE = 2
NW = NCORE * NTILE
EP = 163840
EW = EP // NW
RPT = NPAD // NTILE
RB = 632
GR = NPAD // RB


def _sc_mesh():
    return plsc.VectorSubcoreMesh(core_axis_name="c", subcore_axis_name="s")


def _deg_pass(dstR, cones, zrows):
    DCH2 = 128
    NIT = EW // DCH2

    @functools.partial(
        pl.kernel,
        out_type=jax.ShapeDtypeStruct((NCORE, NPAD, 128), jnp.float32),
        mesh=_sc_mesh(),
        scratch_types=[
            pltpu.VMEM_SHARED((NPAD, 128), jnp.float32),
            pltpu.VMEM((NIT, DCH2), jnp.int32),
            pltpu.VMEM((DCH2, 128), jnp.float32),
            pltpu.SemaphoreType.DMA,
        ],
    )
    def k(dst_hbm, cones_hbm, z_hbm, out_hbm, acc, didx, ones_v, sems):
        c = lax.axis_index("c")
        s = lax.axis_index("s")
        wid = c * NTILE + s
        pltpu.sync_copy(cones_hbm, ones_v)
        pltpu.sync_copy(dst_hbm.at[pl.ds(wid * NIT, NIT)], didx)
        pltpu.sync_copy(z_hbm, acc.at[pl.ds(s * RPT, RPT)])
        plsc.subcore_barrier()

        def drain_s():
            pltpu.make_async_copy(z_hbm.at[pl.ds(0, DCH2)], ones_v, sems).wait()

        def body(it, carry):
            pltpu.async_copy(ones_v, acc.at[didx.at[it]], sems, add=True)

            @pl.when(it >= 2)
            def _():
                drain_s()

            return carry

        lax.fori_loop(0, NIT, body, 0)
        drain_s()
        drain_s()
        plsc.subcore_barrier()
        pltpu.sync_copy(acc.at[pl.ds(s * RPT, RPT)],
                        out_hbm.at[c, pl.ds(s * RPT, RPT)])

    return k(dstR, cones, zrows)


def _prop_pass(gtab, htab, eafl, srcp, dstp, zrows, O):
    HWID = DE * O
    CH = 64 if O <= 64 else 32
    NIT = EW // CH

    @functools.partial(
        pl.kernel,
        out_type=(jax.ShapeDtypeStruct((NCORE, NPAD, 128), jnp.float32),
                  jax.ShapeDtypeStruct((NCORE, NPAD, 128), jnp.float32)),
        mesh=_sc_mesh(),
        scratch_types=[
            pltpu.VMEM_SHARED((NPAD, 128), jnp.float32),
            pltpu.VMEM((EW + CH,), jnp.int32),
            pltpu.VMEM((CH,), jnp.int32),
            pltpu.VMEM((CH,), jnp.int32),
            pltpu.VMEM((CH, 128), jnp.float32),
            pltpu.VMEM((CH, 128), jnp.float32),
            pltpu.VMEM((CH, HWID), jnp.float32),
            pltpu.VMEM((CH * 16,), jnp.float32),
            pltpu.SemaphoreType.DMA,
            pltpu.SemaphoreType.DMA,
        ],
    )
    def k(g_hbm, h_hbm, ea_hbm, src_hbm, dst_hbm, z_hbm, pg_hbm, pe_hbm,
          acc, sidx, didx0, didx1, grow0, grow1, hrow, eav, semg, sems):
        c = lax.axis_index("c")
        s = lax.axis_index("s")
        wid = c * NTILE + s
        ebase = wid * EW
        pltpu.sync_copy(src_hbm.at[pl.ds(ebase, EW + CH)], sidx)

        zg = z_hbm.at[pl.ds(0, CH)]

        def gslice(it):
            return g_hbm.at[sidx.at[pl.ds(it * CH, CH)]]

        def hslice(it):
            return h_hbm.at[sidx.at[pl.ds(it * CH, CH)]]

        def load_didx(it, buf):
            pltpu.sync_copy(dst_hbm.at[pl.ds(ebase + it * CH, CH)], buf)

        def load_eav(it):
            pltpu.sync_copy(ea_hbm.at[pl.ds((ebase + it * CH) * 16, CH * 16)],
                            eav)

        def drain_g(buf):
            pltpu.make_async_copy(zg, buf, semg).wait()

        def drain_gh():
            pltpu.make_async_copy(h_hbm.at[pl.ds(0, CH)], hrow, semg).wait()

        def drain_s(buf):
            pltpu.make_async_copy(zg, buf, sems).wait()

        def zero_acc():
            pltpu.sync_copy(z_hbm, acc.at[pl.ds(s * RPT, RPT)])

        zero_acc()
        plsc.subcore_barrier()
        pltpu.async_copy(gslice(0), grow0, semg)

        def gcn_pair(t, carry):
            it = 2 * t
            drain_g(grow0)

            @pl.when(t > 0)
            def _():
                drain_s(grow1)

            pltpu.async_copy(gslice(it + 1), grow1, semg)
            load_didx(it, didx0)
            pltpu.async_copy(grow0, acc.at[didx0], sems, add=True)
            drain_g(grow1)
            drain_s(grow0)
            pltpu.async_copy(gslice(it + 2), grow0, semg)
            load_didx(it + 1, didx1)
            pltpu.async_copy(grow1, acc.at[didx1], sems, add=True)
            return carry

        lax.fori_loop(0, NIT // 2, gcn_pair, 0)
        drain_g(grow0)
        drain_s(grow1)
        plsc.subcore_barrier()
        pltpu.sync_copy(acc.at[pl.ds(s * RPT, RPT)],
                        pg_hbm.at[c, pl.ds(s * RPT, RPT)])

        zero_acc()
        if O < 128:
            pltpu.sync_copy(zg, grow0)
            pltpu.sync_copy(zg, grow1)
        plsc.subcore_barrier()
        pltpu.async_copy(hslice(0), hrow, semg)

        def compute_msg(gbuf):
            def edge(i, ecarry):
                ev = eav[pl.ds(i * 16, 16)]
                e0 = ev[0]
                e1 = ev[1]
                e2 = ev[2]
                e3 = ev[3]
                for j in range(O // 16):
                    v = (e0 * hrow[i, pl.ds(j * 16, 16)]
                         + e1 * hrow[i, pl.ds(O + j * 16, 16)]
                         + e2 * hrow[i, pl.ds(2 * O + j * 16, 16)]
                         + e3 * hrow[i, pl.ds(3 * O + j * 16, 16)])
                    gbuf[i, pl.ds(j * 16, 16)] = v
                return ecarry

            lax.fori_loop(0, CH, edge, 0)

        def ecc_pair(t, carry):
            it = 2 * t
            drain_gh()
            load_eav(it)
            compute_msg(grow0)

            @pl.when(t > 0)
            def _():
                drain_s(grow1)

            pltpu.async_copy(hslice(it + 1), hrow, semg)
            load_didx(it, didx0)
            pltpu.async_copy(grow0, acc.at[didx0], sems, add=True)
            drain_gh()
            load_eav(it + 1)
            compute_msg(grow1)
            drain_s(grow0)
            pltpu.async_copy(hslice(it + 2), hrow, semg)
            load_didx(it + 1, didx1)
            pltpu.async_copy(grow1, acc.at[didx1], sems, add=True)
            return carry

        lax.fori_loop(0, NIT // 2, ecc_pair, 0)
        drain_gh()
        drain_s(grow1)
        plsc.subcore_barrier()
        pltpu.sync_copy(acc.at[pl.ds(s * RPT, RPT)],
                        pe_hbm.at[c, pl.ds(s * RPT, RPT)])

    return k(gtab, htab, eafl, srcp, dstp, zrows)


def _full(shape):
    return pl.BlockSpec(shape, lambda i: tuple(0 for _ in shape))


def _tc1(degp, xp, w1, k1f, r1w):
    def body(degp_ref, x_ref, w1_ref, k1_ref, r1_ref,
             dis_ref, g1_ref, h1_ref, rr1_ref):
        d = degp_ref[0] + degp_ref[1]
        dis = lax.rsqrt(jnp.maximum(d[:, 0:1], 1.0))
        dis_ref[...] = dis
        xb = x_ref[...]
        g1_ref[...] = dis * jnp.dot(xb, w1_ref[...],
                                    preferred_element_type=jnp.float32)
        h1_ref[...] = jnp.dot(xb, k1_ref[...],
                              preferred_element_type=jnp.float32)
        rr1_ref[...] = jnp.dot(xb, r1_ref[...],
                               preferred_element_type=jnp.float32)

    return pl.pallas_call(
        body,
        grid=(GR,),
        in_specs=[
            pl.BlockSpec((NCORE, RB, 128), lambda i: (0, i, 0)),
            pl.BlockSpec((RB, F), lambda i: (i, 0)),
            _full((F, 128)),
            _full((F, DE * NN)),
            _full((F, NN)),
        ],
        out_specs=[
            pl.BlockSpec((RB, 1), lambda i: (i, 0)),
            pl.BlockSpec((RB, 128), lambda i: (i, 0)),
            pl.BlockSpec((RB, DE * NN), lambda i: (i, 0)),
            pl.BlockSpec((RB, NN), lambda i: (i, 0)),
        ],
        out_shape=[
            jax.ShapeDtypeStruct((NPAD, 1), jnp.float32),
            jax.ShapeDtypeStruct((NPAD, 128), jnp.float32),
            jax.ShapeDtypeStruct((NPAD, DE * NN), jnp.float32),
            jax.ShapeDtypeStruct((NPAD, NN), jnp.float32),
        ],
    )(degp, xp, w1, k1f, r1w)


def _tc2(p1g, p1e, rr1, dis, b1, be1, wb1, kb1f, rb1w):
    def body(pg_ref, pe_ref, rr1_ref, dis_ref, b1_ref, be1_ref,
             wb1_ref, kb1_ref, rb1_ref,
             o1_ref, o2_ref, g2_ref, h2_ref, rr2_ref):
        dis = dis_ref[...]
        o1 = jnp.maximum((pg_ref[0, :, :NN] + pg_ref[1, :, :NN]) * dis
                         + b1_ref[...], 0.0)
        o2 = jnp.maximum(pe_ref[0, :, :NN] + pe_ref[1, :, :NN]
                         + rr1_ref[...] + be1_ref[...], 0.0)
        o1_ref[...] = o1
        o2_ref[...] = o2
        g2_ref[...] = dis * jnp.dot(o1, wb1_ref[...],
                                    preferred_element_type=jnp.float32)
        h2_ref[...] = jnp.dot(o2, kb1_ref[...],
                              preferred_element_type=jnp.float32)
        rr2_ref[...] = jnp.dot(o2, rb1_ref[...],
                               preferred_element_type=jnp.float32)

    return pl.pallas_call(
        body,
        grid=(GR,),
        in_specs=[
            pl.BlockSpec((NCORE, RB, 128), lambda i: (0, i, 0)),
            pl.BlockSpec((NCORE, RB, 128), lambda i: (0, i, 0)),
            pl.BlockSpec((RB, NN), lambda i: (i, 0)),
            pl.BlockSpec((RB, 1), lambda i: (i, 0)),
            _full((1, NN)),
            _full((1, NN)),
            _full((NN, 128)),
            _full((NN, DE * NN)),
            _full((NN, NN)),
        ],
        out_specs=[
            pl.BlockSpec((RB, NN), lambda i: (i, 0)),
            pl.BlockSpec((RB, NN), lambda i: (i, 0)),
            pl.BlockSpec((RB, 128), lambda i: (i, 0)),
            pl.BlockSpec((RB, DE * NN), lambda i: (i, 0)),
            pl.BlockSpec((RB, NN), lambda i: (i, 0)),
        ],
        out_shape=[
            jax.ShapeDtypeStruct((NPAD, NN), jnp.float32),
            jax.ShapeDtypeStruct((NPAD, NN), jnp.float32),
            jax.ShapeDtypeStruct((NPAD, 128), jnp.float32),
            jax.ShapeDtypeStruct((NPAD, DE * NN), jnp.float32),
            jax.ShapeDtypeStruct((NPAD, NN), jnp.float32),
        ],
    )(p1g, p1e, rr1, dis, b1, be1, wb1, kb1f, rb1w)


def _tc3(p2g, p2e, o1, o2, rr2, dis, bb1, beb1, wb2, kb2f, rb2w, proj1, proj2):
    NO = 2 * NN

    def body(pg_ref, pe_ref, o1_ref, o2_ref, rr2_ref, dis_ref,
             bb1_ref, beb1_ref, wb2_ref, kb2_ref, rb2_ref, pj1_ref, pj2_ref,
             g3_ref, h3_ref, rr3_ref, pr1_ref, pr2_ref):
        dis = dis_ref[...]
        o1b = jnp.maximum((pg_ref[0, :, :NN] + pg_ref[1, :, :NN]) * dis
                          + bb1_ref[...] + o1_ref[...], 0.0)
        o2b = jnp.maximum(pe_ref[0, :, :NN] + pe_ref[1, :, :NN]
                          + rr2_ref[...] + beb1_ref[...] + o2_ref[...], 0.0)
        g3_ref[...] = dis * jnp.dot(o1b, wb2_ref[...],
                                    preferred_element_type=jnp.float32)
        h3_ref[...] = jnp.dot(o2b, kb2_ref[...],
                              preferred_element_type=jnp.float32)
        rr3_ref[...] = jnp.dot(o2b, rb2_ref[...],
                               preferred_element_type=jnp.float32)
        pr1_ref[...] = jnp.dot(o1b, pj1_ref[...],
                               preferred_element_type=jnp.float32)
        pr2_ref[...] = jnp.dot(o2b, pj2_ref[...],
                               preferred_element_type=jnp.float32)

    return pl.pallas_call(
        body,
        grid=(GR,),
        in_specs=[
            pl.BlockSpec((NCORE, RB, 128), lambda i: (0, i, 0)),
            pl.BlockSpec((NCORE, RB, 128), lambda i: (0, i, 0)),
            pl.BlockSpec((RB, NN), lambda i: (i, 0)),
            pl.BlockSpec((RB, NN), lambda i: (i, 0)),
            pl.BlockSpec((RB, NN), lambda i: (i, 0)),
            pl.BlockSpec((RB, 1), lambda i: (i, 0)),
            _full((1, NN)),
            _full((1, NN)),
            _full((NN, NO)),
            _full((NN, DE * NO)),
            _full((NN, NO)),
            _full((NN, NO)),
            _full((NN, NO)),
        ],
        out_specs=[
            pl.BlockSpec((RB, NO), lambda i: (i, 0)),
            pl.BlockSpec((RB, DE * NO), lambda i: (i, 0)),
            pl.BlockSpec((RB, NO), lambda i: (i, 0)),
            pl.BlockSpec((RB, NO), lambda i: (i, 0)),
            pl.BlockSpec((RB, NO), lambda i: (i, 0)),
        ],
        out_shape=[
            jax.ShapeDtypeStruct((NPAD, NO), jnp.float32),
            jax.ShapeDtypeStruct((NPAD, DE * NO), jnp.float32),
            jax.ShapeDtypeStruct((NPAD, NO), jnp.float32),
            jax.ShapeDtypeStruct((NPAD, NO), jnp.float32),
            jax.ShapeDtypeStruct((NPAD, NO), jnp.float32),
        ],
    )(p2g, p2e, o1, o2, rr2, dis, bb1, beb1, wb2, kb2f, rb2w, proj1, proj2)


def _tc4(p3g, p3e, rr3, pr1, pr2, dis, bb2, beb2, gids,
         d1w, d1b, d2w, d2b, d3wp, d3bp):
    NO = 2 * NN

    def body(pg_ref, pe_ref, rr3_ref, pr1_ref, pr2_ref, dis_ref,
             bb2_ref, beb2_ref, gid_ref, d1w_ref, d1b_ref, d2w_ref, d2b_ref,
             d3w_ref, d3b_ref, out_ref, acc_ref):
        i = pl.program_id(0)

        @pl.when(i == 0)
        def _():
            acc_ref[...] = jnp.zeros_like(acc_ref)

        dis = dis_ref[...]
        o1c = jnp.maximum((pg_ref[0] + pg_ref[1]) * dis + bb2_ref[...]
                          + pr1_ref[...], 0.0)
        o2c = jnp.maximum(pe_ref[0] + pe_ref[1] + rr3_ref[...] + beb2_ref[...]
                          + pr2_ref[...], 0.0)
        seg = (gid_ref[...] == lax.broadcasted_iota(jnp.int32, (RB, NG), 1)
               ).astype(jnp.float32)
        dn = (((0,), (0,)), ((), ()))
        acc_ref[:, 0:NO] += lax.dot_general(
            seg, o1c, dn, preferred_element_type=jnp.float32)
        acc_ref[:, NO:2 * NO] += lax.dot_general(
            seg, o2c, dn, preferred_element_type=jnp.float32)

        @pl.when(i == GR - 1)
        def _():
            p = acc_ref[...]
            h1 = jnp.maximum(
                jnp.dot(p, d1w_ref[...], preferred_element_type=jnp.float32)
                + d1b_ref[...], 0.0)
            h2 = jnp.maximum(
                jnp.dot(h1, d2w_ref[...], preferred_element_type=jnp.float32)
                + d2b_ref[...], 0.0)
            lg = jnp.dot(h2, d3w_ref[...],
                         preferred_element_type=jnp.float32) + d3b_ref[...]
            mx = jnp.max(lg, axis=1, keepdims=True)
            ex = jnp.exp(lg - mx)
            out_ref[...] = ex / jnp.sum(ex, axis=1, keepdims=True)

    return pl.pallas_call(
        body,
        grid=(GR,),
        in_specs=[
            pl.BlockSpec((NCORE, RB, NO), lambda i: (0, i, 0)),
            pl.BlockSpec((NCORE, RB, NO), lambda i: (0, i, 0)),
            pl.BlockSpec((RB, NO), lambda i: (i, 0)),
            pl.BlockSpec((RB, NO), lambda i: (i, 0)),
            pl.BlockSpec((RB, NO), lambda i: (i, 0)),
            pl.BlockSpec((RB, 1), lambda i: (i, 0)),
            _full((1, NO)),
            _full((1, NO)),
            pl.BlockSpec((RB, 1), lambda i: (i, 0)),
            _full((2 * NO, NN)),
            _full((1, NN)),
            _full((NN, NN // 2)),
            _full((1, NN // 2)),
            _full((NN // 2, 128)),
            _full((1, 128)),
        ],
        out_specs=pl.BlockSpec((NG, 128), lambda i: (0, 0)),
        out_shape=jax.ShapeDtypeStruct((NG, 128), jnp.float32),
        scratch_shapes=[pltpu.VMEM((NG, 2 * NO), jnp.float32)],
    )(p3g, p3e, rr3, pr1, pr2, dis, bb2, beb2, gids,
      d1w, d1b, d2w, d2b, d3wp, d3bp)


def kernel(x, edge_index, edge_attr, graph_ids,
           gcn1_W, gcn1_b, gcnb1_W, gcnb1_b, gcnb2_W, gcnb2_b, gcnb2_proj,
           ecc1_K, ecc1_root, ecc1_b, eccb1_K, eccb1_root, eccb1_b,
           eccb2_K, eccb2_root, eccb2_b, eccb2_proj,
           d1_W, d1_b, d2_W, d2_b, d3_W, d3_b):
    padE = EP - E
    srcp = jnp.concatenate(
        [edge_index[0], jnp.zeros((padE + 128,), jnp.int32)])
    dstp = jnp.concatenate(
        [edge_index[1], jnp.full((padE + 128,), N, jnp.int32)])
    eafl = jnp.pad(edge_attr, ((0, padE), (0, 16 - DE))).reshape(-1)
    dstR128 = dstp[:EP].reshape(-1, 128)
    xp = jnp.pad(x, ((0, NPAD - N), (0, 0)))
    gidsp = jnp.concatenate(
        [graph_ids, jnp.full((NPAD - N,), NG, jnp.int32)]).reshape(NPAD, 1)

    k1f = jnp.transpose(ecc1_K, (1, 0, 2)).reshape(F, DE * NN)
    kb1f = jnp.transpose(eccb1_K, (1, 0, 2)).reshape(NN, DE * NN)
    kb2f = jnp.transpose(eccb2_K, (1, 0, 2)).reshape(NN, DE * 2 * NN)

    cones = jnp.concatenate(
        [jnp.ones((128, 1), jnp.float32), jnp.zeros((128, 127), jnp.float32)],
        axis=1)
    z128 = jnp.zeros((RPT, 128), jnp.float32)
    w1p = jnp.pad(gcn1_W, ((0, 0), (0, 128 - NN)))
    wb1p = jnp.pad(gcnb1_W, ((0, 0), (0, 128 - NN)))

    b1 = gcn1_b.reshape(1, NN)
    bb1 = gcnb1_b.reshape(1, NN)
    bb2 = gcnb2_b.reshape(1, 2 * NN)
    be1 = ecc1_b.reshape(1, NN)
    beb1 = eccb1_b.reshape(1, NN)
    beb2 = eccb2_b.reshape(1, 2 * NN)
    d1b = d1_b.reshape(1, NN)
    d2b = d2_b.reshape(1, NN // 2)
    d3wp = jnp.pad(d3_W, ((0, 0), (0, 128 - NOUT)))
    d3bp = jnp.concatenate(
        [d3_b, jnp.full((128 - NOUT,), -1e30, jnp.float32)]).reshape(1, 128)

    degp = _deg_pass(dstR128, cones, z128)
    dis, g1, h1, rr1 = _tc1(degp, xp, w1p, k1f, ecc1_root)
    p1g, p1e = _prop_pass(g1, h1, eafl, srcp, dstp, z128, NN)
    o1, o2, g2, h2, rr2 = _tc2(p1g, p1e, rr1, dis, b1, be1,
                               wb1p, kb1f, eccb1_root)
    p2g, p2e = _prop_pass(g2, h2, eafl, srcp, dstp, z128, NN)
    g3, h3, rr3, pr1, pr2 = _tc3(p2g, p2e, o1, o2, rr2, dis, bb1, beb1,
                                 gcnb2_W, kb2f, eccb2_root,
                                 gcnb2_proj, eccb2_proj)
    p3g, p3e = _prop_pass(g3, h3, eafl, srcp, dstp, z128, 2 * NN)
    outp = _tc4(p3g, p3e, rr3, pr1, pr2, dis, bb2, beb2, gidsp,
                d1_W, d1b, d2_W, d2b, d3wp, d3bp)
    return outp[:, :NOUT]


# speedup vs baseline: 3.0409x; 1.2326x over previous
import functools

import jax
import jax.numpy as jnp
from jax import lax
from jax.experimental import pallas as pl
from jax.experimental.pallas import tpu as pltpu
from jax.experimental.pallas import tpu_sc as plsc

N = 10000
E = 160000
F = 128
DE = 4
NN = 64
NG = 8
NOUT = 10

NPAD = 10112
NTILE = 16
NCORE = 2
NW = NCORE * NTILE
EP = 163840
EW = EP // NW
RPT = NPAD // NTILE
RB = 632
GR = NPAD // RB


def _sc_mesh():
    return plsc.VectorSubcoreMesh(core_axis_name="c", subcore_axis_name="s")


def _deg_pass(dstR, cones, zrows):
    DCH2 = 128
    NIT = EW // DCH2

    @functools.partial(
        pl.kernel,
        out_type=jax.ShapeDtypeStruct((NCORE, NPAD, 128), jnp.float32),
        mesh=_sc_mesh(),
        scratch_types=[
            pltpu.VMEM_SHARED((NPAD, 128), jnp.float32),
            pltpu.VMEM((NIT, DCH2), jnp.int32),
            pltpu.VMEM((DCH2, 128), jnp.float32),
            pltpu.SemaphoreType.DMA,
        ],
    )
    def k(dst_hbm, cones_hbm, z_hbm, out_hbm, acc, didx, ones_v, sems):
        c = lax.axis_index("c")
        s = lax.axis_index("s")
        wid = c * NTILE + s
        pltpu.sync_copy(cones_hbm, ones_v)
        pltpu.sync_copy(dst_hbm.at[pl.ds(wid * NIT, NIT)], didx)
        pltpu.sync_copy(z_hbm, acc.at[pl.ds(s * RPT, RPT)])
        plsc.subcore_barrier()

        def drain_s():
            pltpu.make_async_copy(z_hbm.at[pl.ds(0, DCH2)], ones_v, sems).wait()

        def body(it, carry):
            pltpu.async_copy(ones_v, acc.at[didx.at[it]], sems, add=True)

            @pl.when(it >= 2)
            def _():
                drain_s()

            return carry

        lax.fori_loop(0, NIT, body, 0)
        drain_s()
        drain_s()
        plsc.subcore_barrier()
        pltpu.sync_copy(acc.at[pl.ds(s * RPT, RPT)],
                        out_hbm.at[c, pl.ds(s * RPT, RPT)])

    return k(dstR, cones, zrows)


def _prop_fused(gtab, htab, eafl, srcp, dstp, zrows):
    O = NN
    CH = 64
    NIT = EW // CH

    @functools.partial(
        pl.kernel,
        out_type=jax.ShapeDtypeStruct((NCORE, NPAD, 128), jnp.float32),
        mesh=_sc_mesh(),
        scratch_types=[
            pltpu.VMEM_SHARED((NPAD, 128), jnp.float32),
            pltpu.VMEM((EW + CH,), jnp.int32),
            pltpu.VMEM((CH,), jnp.int32),
            pltpu.VMEM((CH,), jnp.int32),
            pltpu.VMEM((CH, 128), jnp.float32),
            pltpu.VMEM((CH, 128), jnp.float32),
            pltpu.VMEM((CH, DE * O), jnp.float32),
            pltpu.VMEM((CH * 16,), jnp.float32),
            pltpu.SemaphoreType.DMA,
            pltpu.SemaphoreType.DMA,
            pltpu.SemaphoreType.DMA,
        ],
    )
    def k(g_hbm, h_hbm, ea_hbm, src_hbm, dst_hbm, z_hbm, pc_hbm,
          acc, sidx, didx0, didx1, grow0, grow1, hrow, eav, semg, semh, sems):
        c = lax.axis_index("c")
        s = lax.axis_index("s")
        wid = c * NTILE + s
        ebase = wid * EW
        pltpu.sync_copy(src_hbm.at[pl.ds(ebase, EW + CH)], sidx)

        zg = z_hbm.at[pl.ds(0, CH)]

        def gslice(it):
            return g_hbm.at[sidx.at[pl.ds(it * CH, CH)]]

        def hslice(it):
            return h_hbm.at[sidx.at[pl.ds(it * CH, CH)]]

        def load_didx(it, buf):
            pltpu.sync_copy(dst_hbm.at[pl.ds(ebase + it * CH, CH)], buf)

        def load_eav(it):
            pltpu.sync_copy(ea_hbm.at[pl.ds((ebase + it * CH) * 16, CH * 16)],
                            eav)

        def drain_g(buf):
            pltpu.make_async_copy(zg, buf, semg).wait()

        def drain_gh():
            pltpu.make_async_copy(h_hbm.at[pl.ds(0, CH)], hrow, semh).wait()

        def drain_s(buf):
            pltpu.make_async_copy(zg, buf, sems).wait()

        def compute_msg(gbuf):
            def edge(i, ecarry):
                ev = eav[pl.ds(i * 16, 16)]
                e0 = ev[0]
                e1 = ev[1]
                e2 = ev[2]
                e3 = ev[3]
                for j in range(O // 16):
                    v = (e0 * hrow[i, pl.ds(j * 16, 16)]
                         + e1 * hrow[i, pl.ds(O + j * 16, 16)]
                         + e2 * hrow[i, pl.ds(2 * O + j * 16, 16)]
                         + e3 * hrow[i, pl.ds(3 * O + j * 16, 16)])
                    gbuf[i, pl.ds(O + j * 16, 16)] = v
                return ecarry

            lax.fori_loop(0, CH, edge, 0)

        pltpu.sync_copy(z_hbm, acc.at[pl.ds(s * RPT, RPT)])
        plsc.subcore_barrier()
        pltpu.async_copy(gslice(0), grow0, semg)
        pltpu.async_copy(hslice(0), hrow, semh)

        def pair(t, carry):
            it = 2 * t
            drain_g(grow0)
            drain_gh()
            load_eav(it)
            compute_msg(grow0)

            @pl.when(t > 0)
            def _():
                drain_s(grow1)

            pltpu.async_copy(gslice(it + 1), grow1, semg)
            pltpu.async_copy(hslice(it + 1), hrow, semh)
            load_didx(it, didx0)
            pltpu.async_copy(grow0, acc.at[didx0], sems, add=True)
            drain_g(grow1)
            drain_gh()
            load_eav(it + 1)
            compute_msg(grow1)
            drain_s(grow0)
            pltpu.async_copy(gslice(it + 2), grow0, semg)
            pltpu.async_copy(hslice(it + 2), hrow, semh)
            load_didx(it + 1, didx1)
            pltpu.async_copy(grow1, acc.at[didx1], sems, add=True)
            return carry

        lax.fori_loop(0, NIT // 2, pair, 0)
        drain_g(grow0)
        drain_gh()
        drain_s(grow1)
        plsc.subcore_barrier()
        pltpu.sync_copy(acc.at[pl.ds(s * RPT, RPT)],
                        pc_hbm.at[c, pl.ds(s * RPT, RPT)])

    return k(gtab, htab, eafl, srcp, dstp, zrows)


def _prop_pass(gtab, htab, eafl, srcp, dstp, zrows, O):
    HWID = DE * O
    CH = 64 if O <= 64 else 32
    NIT = EW // CH

    @functools.partial(
        pl.kernel,
        out_type=(jax.ShapeDtypeStruct((NCORE, NPAD, 128), jnp.float32),
                  jax.ShapeDtypeStruct((NCORE, NPAD, 128), jnp.float32)),
        mesh=_sc_mesh(),
        scratch_types=[
            pltpu.VMEM_SHARED((NPAD, 128), jnp.float32),
            pltpu.VMEM((EW + CH,), jnp.int32),
            pltpu.VMEM((CH,), jnp.int32),
            pltpu.VMEM((CH,), jnp.int32),
            pltpu.VMEM((CH, 128), jnp.float32),
            pltpu.VMEM((CH, 128), jnp.float32),
            pltpu.VMEM((CH, HWID), jnp.float32),
            pltpu.VMEM((CH * 16,), jnp.float32),
            pltpu.SemaphoreType.DMA,
            pltpu.SemaphoreType.DMA,
        ],
    )
    def k(g_hbm, h_hbm, ea_hbm, src_hbm, dst_hbm, z_hbm, pg_hbm, pe_hbm,
          acc, sidx, didx0, didx1, grow0, grow1, hrow, eav, semg, sems):
        c = lax.axis_index("c")
        s = lax.axis_index("s")
        wid = c * NTILE + s
        ebase = wid * EW
        pltpu.sync_copy(src_hbm.at[pl.ds(ebase, EW + CH)], sidx)

        zg = z_hbm.at[pl.ds(0, CH)]

        def gslice(it):
            return g_hbm.at[sidx.at[pl.ds(it * CH, CH)]]

        def hslice(it):
            return h_hbm.at[sidx.at[pl.ds(it * CH, CH)]]

        def load_didx(it, buf):
            pltpu.sync_copy(dst_hbm.at[pl.ds(ebase + it * CH, CH)], buf)

        def load_eav(it):
            pltpu.sync_copy(ea_hbm.at[pl.ds((ebase + it * CH) * 16, CH * 16)],
                            eav)

        def drain_g(buf):
            pltpu.make_async_copy(zg, buf, semg).wait()

        def drain_gh():
            pltpu.make_async_copy(h_hbm.at[pl.ds(0, CH)], hrow, semg).wait()

        def drain_s(buf):
            pltpu.make_async_copy(zg, buf, sems).wait()

        def zero_acc():
            pltpu.sync_copy(z_hbm, acc.at[pl.ds(s * RPT, RPT)])

        zero_acc()
        plsc.subcore_barrier()
        pltpu.async_copy(gslice(0), grow0, semg)

        def gcn_pair(t, carry):
            it = 2 * t
            drain_g(grow0)

            @pl.when(t > 0)
            def _():
                drain_s(grow1)

            pltpu.async_copy(gslice(it + 1), grow1, semg)
            load_didx(it, didx0)
            pltpu.async_copy(grow0, acc.at[didx0], sems, add=True)
            drain_g(grow1)
            drain_s(grow0)
            pltpu.async_copy(gslice(it + 2), grow0, semg)
            load_didx(it + 1, didx1)
            pltpu.async_copy(grow1, acc.at[didx1], sems, add=True)
            return carry

        lax.fori_loop(0, NIT // 2, gcn_pair, 0)
        drain_g(grow0)
        drain_s(grow1)
        plsc.subcore_barrier()
        pltpu.sync_copy(acc.at[pl.ds(s * RPT, RPT)],
                        pg_hbm.at[c, pl.ds(s * RPT, RPT)])

        zero_acc()
        if O < 128:
            pltpu.sync_copy(zg, grow0)
            pltpu.sync_copy(zg, grow1)
        plsc.subcore_barrier()
        pltpu.async_copy(hslice(0), hrow, semg)

        def compute_msg(gbuf):
            def edge(i, ecarry):
                ev = eav[pl.ds(i * 16, 16)]
                e0 = ev[0]
                e1 = ev[1]
                e2 = ev[2]
                e3 = ev[3]
                for j in range(O // 16):
                    v = (e0 * hrow[i, pl.ds(j * 16, 16)]
                         + e1 * hrow[i, pl.ds(O + j * 16, 16)]
                         + e2 * hrow[i, pl.ds(2 * O + j * 16, 16)]
                         + e3 * hrow[i, pl.ds(3 * O + j * 16, 16)])
                    gbuf[i, pl.ds(j * 16, 16)] = v
                return ecarry

            lax.fori_loop(0, CH, edge, 0)

        def ecc_pair(t, carry):
            it = 2 * t
            drain_gh()
            load_eav(it)
            compute_msg(grow0)

            @pl.when(t > 0)
            def _():
                drain_s(grow1)

            pltpu.async_copy(hslice(it + 1), hrow, semg)
            load_didx(it, didx0)
            pltpu.async_copy(grow0, acc.at[didx0], sems, add=True)
            drain_gh()
            load_eav(it + 1)
            compute_msg(grow1)
            drain_s(grow0)
            pltpu.async_copy(hslice(it + 2), hrow, semg)
            load_didx(it + 1, didx1)
            pltpu.async_copy(grow1, acc.at[didx1], sems, add=True)
            return carry

        lax.fori_loop(0, NIT // 2, ecc_pair, 0)
        drain_gh()
        drain_s(grow1)
        plsc.subcore_barrier()
        pltpu.sync_copy(acc.at[pl.ds(s * RPT, RPT)],
                        pe_hbm.at[c, pl.ds(s * RPT, RPT)])

    return k(gtab, htab, eafl, srcp, dstp, zrows)


def _full(shape):
    return pl.BlockSpec(shape, lambda i: tuple(0 for _ in shape))


def _tc1(degp, xp, w1, k1f, r1w):
    def body(degp_ref, x_ref, w1_ref, k1_ref, r1_ref,
             dis_ref, g1_ref, h1_ref, rr1_ref):
        d = degp_ref[0] + degp_ref[1]
        dis = lax.rsqrt(jnp.maximum(d[:, 0:1], 1.0))
        dis_ref[...] = dis
        xb = x_ref[...]
        g1_ref[...] = dis * jnp.dot(xb, w1_ref[...],
                                    preferred_element_type=jnp.float32)
        h1_ref[...] = jnp.dot(xb, k1_ref[...],
                              preferred_element_type=jnp.float32)
        rr1_ref[...] = jnp.dot(xb, r1_ref[...],
                               preferred_element_type=jnp.float32)

    return pl.pallas_call(
        body,
        grid=(GR,),
        in_specs=[
            pl.BlockSpec((NCORE, RB, 128), lambda i: (0, i, 0)),
            pl.BlockSpec((RB, F), lambda i: (i, 0)),
            _full((F, 128)),
            _full((F, DE * NN)),
            _full((F, NN)),
        ],
        out_specs=[
            pl.BlockSpec((RB, 1), lambda i: (i, 0)),
            pl.BlockSpec((RB, 128), lambda i: (i, 0)),
            pl.BlockSpec((RB, DE * NN), lambda i: (i, 0)),
            pl.BlockSpec((RB, NN), lambda i: (i, 0)),
        ],
        out_shape=[
            jax.ShapeDtypeStruct((NPAD, 1), jnp.float32),
            jax.ShapeDtypeStruct((NPAD, 128), jnp.float32),
            jax.ShapeDtypeStruct((NPAD, DE * NN), jnp.float32),
            jax.ShapeDtypeStruct((NPAD, NN), jnp.float32),
        ],
    )(degp, xp, w1, k1f, r1w)


def _tc2(p1c, rr1, dis, b1, be1, wb1, kb1f, rb1w):
    def body(pc_ref, rr1_ref, dis_ref, b1_ref, be1_ref,
             wb1_ref, kb1_ref, rb1_ref,
             o1_ref, o2_ref, g2_ref, h2_ref, rr2_ref):
        dis = dis_ref[...]
        o1 = jnp.maximum((pc_ref[0, :, :NN] + pc_ref[1, :, :NN]) * dis
                         + b1_ref[...], 0.0)
        o2 = jnp.maximum(pc_ref[0, :, NN:128] + pc_ref[1, :, NN:128]
                         + rr1_ref[...] + be1_ref[...], 0.0)
        o1_ref[...] = o1
        o2_ref[...] = o2
        g2_ref[...] = dis * jnp.dot(o1, wb1_ref[...],
                                    preferred_element_type=jnp.float32)
        h2_ref[...] = jnp.dot(o2, kb1_ref[...],
                              preferred_element_type=jnp.float32)
        rr2_ref[...] = jnp.dot(o2, rb1_ref[...],
                               preferred_element_type=jnp.float32)

    return pl.pallas_call(
        body,
        grid=(GR,),
        in_specs=[
            pl.BlockSpec((NCORE, RB, 128), lambda i: (0, i, 0)),
            pl.BlockSpec((RB, NN), lambda i: (i, 0)),
            pl.BlockSpec((RB, 1), lambda i: (i, 0)),
            _full((1, NN)),
            _full((1, NN)),
            _full((NN, 128)),
            _full((NN, DE * NN)),
            _full((NN, NN)),
        ],
        out_specs=[
            pl.BlockSpec((RB, NN), lambda i: (i, 0)),
            pl.BlockSpec((RB, NN), lambda i: (i, 0)),
            pl.BlockSpec((RB, 128), lambda i: (i, 0)),
            pl.BlockSpec((RB, DE * NN), lambda i: (i, 0)),
            pl.BlockSpec((RB, NN), lambda i: (i, 0)),
        ],
        out_shape=[
            jax.ShapeDtypeStruct((NPAD, NN), jnp.float32),
            jax.ShapeDtypeStruct((NPAD, NN), jnp.float32),
            jax.ShapeDtypeStruct((NPAD, 128), jnp.float32),
            jax.ShapeDtypeStruct((NPAD, DE * NN), jnp.float32),
            jax.ShapeDtypeStruct((NPAD, NN), jnp.float32),
        ],
    )(p1c, rr1, dis, b1, be1, wb1, kb1f, rb1w)


def _tc3(p2c, o1, o2, rr2, dis, bb1, beb1, wb2, kb2f, rb2w, proj1, proj2):
    NO = 2 * NN

    def body(pc_ref, o1_ref, o2_ref, rr2_ref, dis_ref,
             bb1_ref, beb1_ref, wb2_ref, kb2_ref, rb2_ref, pj1_ref, pj2_ref,
             g3_ref, h3_ref, rr3_ref, pr1_ref, pr2_ref):
        dis = dis_ref[...]
        o1b = jnp.maximum((pc_ref[0, :, :NN] + pc_ref[1, :, :NN]) * dis
                          + bb1_ref[...] + o1_ref[...], 0.0)
        o2b = jnp.maximum(pc_ref[0, :, NN:128] + pc_ref[1, :, NN:128]
                          + rr2_ref[...] + beb1_ref[...] + o2_ref[...], 0.0)
        g3_ref[...] = dis * jnp.dot(o1b, wb2_ref[...],
                                    preferred_element_type=jnp.float32)
        h3_ref[...] = jnp.dot(o2b, kb2_ref[...],
                              preferred_element_type=jnp.float32)
        rr3_ref[...] = jnp.dot(o2b, rb2_ref[...],
                               preferred_element_type=jnp.float32)
        pr1_ref[...] = jnp.dot(o1b, pj1_ref[...],
                               preferred_element_type=jnp.float32)
        pr2_ref[...] = jnp.dot(o2b, pj2_ref[...],
                               preferred_element_type=jnp.float32)

    return pl.pallas_call(
        body,
        grid=(GR,),
        in_specs=[
            pl.BlockSpec((NCORE, RB, 128), lambda i: (0, i, 0)),
            pl.BlockSpec((RB, NN), lambda i: (i, 0)),
            pl.BlockSpec((RB, NN), lambda i: (i, 0)),
            pl.BlockSpec((RB, NN), lambda i: (i, 0)),
            pl.BlockSpec((RB, 1), lambda i: (i, 0)),
            _full((1, NN)),
            _full((1, NN)),
            _full((NN, NO)),
            _full((NN, DE * NO)),
            _full((NN, NO)),
            _full((NN, NO)),
            _full((NN, NO)),
        ],
        out_specs=[
            pl.BlockSpec((RB, NO), lambda i: (i, 0)),
            pl.BlockSpec((RB, DE * NO), lambda i: (i, 0)),
            pl.BlockSpec((RB, NO), lambda i: (i, 0)),
            pl.BlockSpec((RB, NO), lambda i: (i, 0)),
            pl.BlockSpec((RB, NO), lambda i: (i, 0)),
        ],
        out_shape=[
            jax.ShapeDtypeStruct((NPAD, NO), jnp.float32),
            jax.ShapeDtypeStruct((NPAD, DE * NO), jnp.float32),
            jax.ShapeDtypeStruct((NPAD, NO), jnp.float32),
            jax.ShapeDtypeStruct((NPAD, NO), jnp.float32),
            jax.ShapeDtypeStruct((NPAD, NO), jnp.float32),
        ],
    )(p2c, o1, o2, rr2, dis, bb1, beb1, wb2, kb2f, rb2w, proj1, proj2)


def _tc4(p3g, p3e, rr3, pr1, pr2, dis, bb2, beb2, gids,
         d1w, d1b, d2w, d2b, d3wp, d3bp):
    NO = 2 * NN

    def body(pg_ref, pe_ref, rr3_ref, pr1_ref, pr2_ref, dis_ref,
             bb2_ref, beb2_ref, gid_ref, d1w_ref, d1b_ref, d2w_ref, d2b_ref,
             d3w_ref, d3b_ref, out_ref, acc_ref):
        i = pl.program_id(0)

        @pl.when(i == 0)
        def _():
            acc_ref[...] = jnp.zeros_like(acc_ref)

        dis = dis_ref[...]
        o1c = jnp.maximum((pg_ref[0] + pg_ref[1]) * dis + bb2_ref[...]
                          + pr1_ref[...], 0.0)
        o2c = jnp.maximum(pe_ref[0] + pe_ref[1] + rr3_ref[...] + beb2_ref[...]
                          + pr2_ref[...], 0.0)
        seg = (gid_ref[...] == lax.broadcasted_iota(jnp.int32, (RB, NG), 1)
               ).astype(jnp.float32)
        dn = (((0,), (0,)), ((), ()))
        acc_ref[:, 0:NO] += lax.dot_general(
            seg, o1c, dn, preferred_element_type=jnp.float32)
        acc_ref[:, NO:2 * NO] += lax.dot_general(
            seg, o2c, dn, preferred_element_type=jnp.float32)

        @pl.when(i == GR - 1)
        def _():
            p = acc_ref[...]
            h1 = jnp.maximum(
                jnp.dot(p, d1w_ref[...], preferred_element_type=jnp.float32)
                + d1b_ref[...], 0.0)
            h2 = jnp.maximum(
                jnp.dot(h1, d2w_ref[...], preferred_element_type=jnp.float32)
                + d2b_ref[...], 0.0)
            lg = jnp.dot(h2, d3w_ref[...],
                         preferred_element_type=jnp.float32) + d3b_ref[...]
            mx = jnp.max(lg, axis=1, keepdims=True)
            ex = jnp.exp(lg - mx)
            out_ref[...] = ex / jnp.sum(ex, axis=1, keepdims=True)

    return pl.pallas_call(
        body,
        grid=(GR,),
        in_specs=[
            pl.BlockSpec((NCORE, RB, NO), lambda i: (0, i, 0)),
            pl.BlockSpec((NCORE, RB, NO), lambda i: (0, i, 0)),
            pl.BlockSpec((RB, NO), lambda i: (i, 0)),
            pl.BlockSpec((RB, NO), lambda i: (i, 0)),
            pl.BlockSpec((RB, NO), lambda i: (i, 0)),
            pl.BlockSpec((RB, 1), lambda i: (i, 0)),
            _full((1, NO)),
            _full((1, NO)),
            pl.BlockSpec((RB, 1), lambda i: (i, 0)),
            _full((2 * NO, NN)),
            _full((1, NN)),
            _full((NN, NN // 2)),
            _full((1, NN // 2)),
            _full((NN // 2, 128)),
            _full((1, 128)),
        ],
        out_specs=pl.BlockSpec((NG, 128), lambda i: (0, 0)),
        out_shape=jax.ShapeDtypeStruct((NG, 128), jnp.float32),
        scratch_shapes=[pltpu.VMEM((NG, 2 * NO), jnp.float32)],
    )(p3g, p3e, rr3, pr1, pr2, dis, bb2, beb2, gids,
      d1w, d1b, d2w, d2b, d3wp, d3bp)


def kernel(x, edge_index, edge_attr, graph_ids,
           gcn1_W, gcn1_b, gcnb1_W, gcnb1_b, gcnb2_W, gcnb2_b, gcnb2_proj,
           ecc1_K, ecc1_root, ecc1_b, eccb1_K, eccb1_root, eccb1_b,
           eccb2_K, eccb2_root, eccb2_b, eccb2_proj,
           d1_W, d1_b, d2_W, d2_b, d3_W, d3_b):
    padE = EP - E
    srcp = jnp.concatenate(
        [edge_index[0], jnp.zeros((padE + 128,), jnp.int32)])
    dstp = jnp.concatenate(
        [edge_index[1], jnp.full((padE + 128,), N, jnp.int32)])
    eafl = jnp.pad(edge_attr, ((0, padE), (0, 16 - DE))).reshape(-1)
    dstR128 = dstp[:EP].reshape(-1, 128)
    xp = jnp.pad(x, ((0, NPAD - N), (0, 0)))
    gidsp = jnp.concatenate(
        [graph_ids, jnp.full((NPAD - N,), NG, jnp.int32)]).reshape(NPAD, 1)

    k1f = jnp.transpose(ecc1_K, (1, 0, 2)).reshape(F, DE * NN)
    kb1f = jnp.transpose(eccb1_K, (1, 0, 2)).reshape(NN, DE * NN)
    kb2f = jnp.transpose(eccb2_K, (1, 0, 2)).reshape(NN, DE * 2 * NN)

    cones = jnp.concatenate(
        [jnp.ones((128, 1), jnp.float32), jnp.zeros((128, 127), jnp.float32)],
        axis=1)
    z128 = jnp.zeros((RPT, 128), jnp.float32)
    w1p = jnp.pad(gcn1_W, ((0, 0), (0, 128 - NN)))
    wb1p = jnp.pad(gcnb1_W, ((0, 0), (0, 128 - NN)))

    b1 = gcn1_b.reshape(1, NN)
    bb1 = gcnb1_b.reshape(1, NN)
    bb2 = gcnb2_b.reshape(1, 2 * NN)
    be1 = ecc1_b.reshape(1, NN)
    beb1 = eccb1_b.reshape(1, NN)
    beb2 = eccb2_b.reshape(1, 2 * NN)
    d1b = d1_b.reshape(1, NN)
    d2b = d2_b.reshape(1, NN // 2)
    d3wp = jnp.pad(d3_W, ((0, 0), (0, 128 - NOUT)))
    d3bp = jnp.concatenate(
        [d3_b, jnp.full((128 - NOUT,), -1e30, jnp.float32)]).reshape(1, 128)

    degp = _deg_pass(dstR128, cones, z128)
    dis, g1, h1, rr1 = _tc1(degp, xp, w1p, k1f, ecc1_root)
    p1c = _prop_fused(g1, h1, eafl, srcp, dstp, z128)
    o1, o2, g2, h2, rr2 = _tc2(p1c, rr1, dis, b1, be1,
                               wb1p, kb1f, eccb1_root)
    p2c = _prop_fused(g2, h2, eafl, srcp, dstp, z128)
    g3, h3, rr3, pr1, pr2 = _tc3(p2c, o1, o2, rr2, dis, bb1, beb1,
                                 gcnb2_W, kb2f, eccb2_root,
                                 gcnb2_proj, eccb2_proj)
    p3g, p3e = _prop_pass(g3, h3, eafl, srcp, dstp, z128, 2 * NN)
    outp = _tc4(p3g, p3e, rr3, pr1, pr2, dis, bb2, beb2, gidsp,
                d1_W, d1b, d2_W, d2b, d3wp, d3bp)
    return outp[:, :NOUT]


# speedup vs baseline: 3.1067x; 1.0216x over previous
import functools

import jax
import jax.numpy as jnp
from jax import lax
from jax.experimental import pallas as pl
from jax.experimental.pallas import tpu as pltpu
from jax.experimental.pallas import tpu_sc as plsc

N = 10000
E = 160000
F = 128
DE = 4
NN = 64
NG = 8
NOUT = 10

NPAD = 10112
NTILE = 16
NCORE = 2
NW = NCORE * NTILE
EP = 163840
EW = EP // NW
RPT = NPAD // NTILE
RB = 632
GR = NPAD // RB


def _sc_mesh():
    return plsc.VectorSubcoreMesh(core_axis_name="c", subcore_axis_name="s")


def _deg_pass(dstR, cones, zrows):
    DCH2 = 128
    NIT = EW // DCH2

    @functools.partial(
        pl.kernel,
        out_type=jax.ShapeDtypeStruct((NCORE, NPAD, 128), jnp.float32),
        mesh=_sc_mesh(),
        scratch_types=[
            pltpu.VMEM_SHARED((NPAD, 128), jnp.float32),
            pltpu.VMEM((NIT, DCH2), jnp.int32),
            pltpu.VMEM((DCH2, 128), jnp.float32),
            pltpu.SemaphoreType.DMA,
        ],
    )
    def k(dst_hbm, cones_hbm, z_hbm, out_hbm, acc, didx, ones_v, sems):
        c = lax.axis_index("c")
        s = lax.axis_index("s")
        wid = c * NTILE + s
        pltpu.sync_copy(cones_hbm, ones_v)
        pltpu.sync_copy(dst_hbm.at[pl.ds(wid * NIT, NIT)], didx)
        pltpu.sync_copy(z_hbm, acc.at[pl.ds(s * RPT, RPT)])
        plsc.subcore_barrier()

        def drain_s():
            pltpu.make_async_copy(z_hbm.at[pl.ds(0, DCH2)], ones_v, sems).wait()

        def body(it, carry):
            pltpu.async_copy(ones_v, acc.at[didx.at[it]], sems, add=True)

            @pl.when(it >= 2)
            def _():
                drain_s()

            return carry

        lax.fori_loop(0, NIT, body, 0)
        drain_s()
        drain_s()
        plsc.subcore_barrier()
        pltpu.sync_copy(acc.at[pl.ds(s * RPT, RPT)],
                        out_hbm.at[c, pl.ds(s * RPT, RPT)])

    return k(dstR, cones, zrows)


def _prop_fused(gtab, htab, eafl, srcp, dstp, zrows):
    O = NN
    CH = 64
    NIT = EW // CH

    @functools.partial(
        pl.kernel,
        out_type=jax.ShapeDtypeStruct((NCORE, NPAD, 128), jnp.float32),
        mesh=_sc_mesh(),
        scratch_types=[
            pltpu.VMEM_SHARED((NPAD, 128), jnp.float32),
            pltpu.VMEM((EW + CH,), jnp.int32),
            pltpu.VMEM((CH,), jnp.int32),
            pltpu.VMEM((CH,), jnp.int32),
            pltpu.VMEM((CH, 128), jnp.float32),
            pltpu.VMEM((CH, 128), jnp.float32),
            pltpu.VMEM((CH, DE * O), jnp.float32),
            pltpu.VMEM((CH * 16,), jnp.float32),
            pltpu.VMEM((CH * 16,), jnp.float32),
            pltpu.SemaphoreType.DMA,
            pltpu.SemaphoreType.DMA,
            pltpu.SemaphoreType.DMA,
            pltpu.SemaphoreType.DMA,
            pltpu.SemaphoreType.DMA,
        ],
    )
    def k(g_hbm, h_hbm, ea_hbm, src_hbm, dst_hbm, z_hbm, pc_hbm,
          acc, sidx, didx0, didx1, grow0, grow1, hrow, eav0, eav1,
          semg, semh, sems, semd, seme):
        c = lax.axis_index("c")
        s = lax.axis_index("s")
        wid = c * NTILE + s
        ebase = wid * EW
        pltpu.sync_copy(src_hbm.at[pl.ds(ebase, EW + CH)], sidx)

        zg = z_hbm.at[pl.ds(0, CH)]

        def gslice(it):
            return g_hbm.at[sidx.at[pl.ds(it * CH, CH)]]

        def hslice(it):
            return h_hbm.at[sidx.at[pl.ds(it * CH, CH)]]

        def fetch_didx(it, buf):
            pltpu.async_copy(dst_hbm.at[pl.ds(ebase + it * CH, CH)], buf,
                             semd)

        def fetch_eav(it, buf):
            pltpu.async_copy(ea_hbm.at[pl.ds((ebase + it * CH) * 16, CH * 16)],
                             buf, seme)

        def drain_g(buf):
            pltpu.make_async_copy(zg, buf, semg).wait()

        def drain_gh():
            pltpu.make_async_copy(h_hbm.at[pl.ds(0, CH)], hrow, semh).wait()

        def drain_s(buf):
            pltpu.make_async_copy(zg, buf, sems).wait()

        def drain_d():
            pltpu.make_async_copy(dst_hbm.at[pl.ds(0, CH)], didx0, semd).wait()

        def drain_e():
            pltpu.make_async_copy(ea_hbm.at[pl.ds(0, CH * 16)], eav0,
                                  seme).wait()

        def compute_msg(eav, gbuf):
            def edge(i, ecarry):
                ev = eav[pl.ds(i * 16, 16)]
                e0 = ev[0]
                e1 = ev[1]
                e2 = ev[2]
                e3 = ev[3]
                for j in range(O // 16):
                    v = (e0 * hrow[i, pl.ds(j * 16, 16)]
                         + e1 * hrow[i, pl.ds(O + j * 16, 16)]
                         + e2 * hrow[i, pl.ds(2 * O + j * 16, 16)]
                         + e3 * hrow[i, pl.ds(3 * O + j * 16, 16)])
                    gbuf[i, pl.ds(O + j * 16, 16)] = v
                return ecarry

            lax.fori_loop(0, CH, edge, 0)

        pltpu.sync_copy(z_hbm, acc.at[pl.ds(s * RPT, RPT)])
        plsc.subcore_barrier()
        pltpu.async_copy(gslice(0), grow0, semg)
        pltpu.async_copy(hslice(0), hrow, semh)
        fetch_didx(0, didx0)
        fetch_eav(0, eav0)

        def pair(t, carry):
            it = 2 * t
            drain_g(grow0)
            drain_gh()
            fetch_eav(it + 1, eav1)
            drain_e()
            compute_msg(eav0, grow0)

            @pl.when(t > 0)
            def _():
                drain_s(grow1)

            pltpu.async_copy(gslice(it + 1), grow1, semg)
            pltpu.async_copy(hslice(it + 1), hrow, semh)
            fetch_didx(it + 1, didx1)
            drain_d()
            pltpu.async_copy(grow0, acc.at[didx0], sems, add=True)
            drain_g(grow1)
            drain_gh()
            fetch_eav(it + 2, eav0)
            drain_e()
            compute_msg(eav1, grow1)
            drain_s(grow0)
            pltpu.async_copy(gslice(it + 2), grow0, semg)
            pltpu.async_copy(hslice(it + 2), hrow, semh)
            fetch_didx(it + 2, didx0)
            drain_d()
            pltpu.async_copy(grow1, acc.at[didx1], sems, add=True)
            return carry

        lax.fori_loop(0, NIT // 2, pair, 0)
        drain_g(grow0)
        drain_gh()
        drain_d()
        drain_e()
        drain_s(grow1)
        plsc.subcore_barrier()
        pltpu.sync_copy(acc.at[pl.ds(s * RPT, RPT)],
                        pc_hbm.at[c, pl.ds(s * RPT, RPT)])

    return k(gtab, htab, eafl, srcp, dstp, zrows)


def _prop_pass(gtab, htab, eafl, srcp, dstp, zrows, O):
    HWID = DE * O
    CH = 64 if O <= 64 else 32
    NIT = EW // CH

    @functools.partial(
        pl.kernel,
        out_type=(jax.ShapeDtypeStruct((NCORE, NPAD, 128), jnp.float32),
                  jax.ShapeDtypeStruct((NCORE, NPAD, 128), jnp.float32)),
        mesh=_sc_mesh(),
        scratch_types=[
            pltpu.VMEM_SHARED((NPAD, 128), jnp.float32),
            pltpu.VMEM((EW + CH,), jnp.int32),
            pltpu.VMEM((CH,), jnp.int32),
            pltpu.VMEM((CH,), jnp.int32),
            pltpu.VMEM((CH, 128), jnp.float32),
            pltpu.VMEM((CH, 128), jnp.float32),
            pltpu.VMEM((CH, HWID), jnp.float32),
            pltpu.VMEM((CH * 16,), jnp.float32),
            pltpu.VMEM((CH * 16,), jnp.float32),
            pltpu.SemaphoreType.DMA,
            pltpu.SemaphoreType.DMA,
            pltpu.SemaphoreType.DMA,
            pltpu.SemaphoreType.DMA,
        ],
    )
    def k(g_hbm, h_hbm, ea_hbm, src_hbm, dst_hbm, z_hbm, pg_hbm, pe_hbm,
          acc, sidx, didx0, didx1, grow0, grow1, hrow, eav0, eav1,
          semg, sems, semd, seme):
        c = lax.axis_index("c")
        s = lax.axis_index("s")
        wid = c * NTILE + s
        ebase = wid * EW
        pltpu.sync_copy(src_hbm.at[pl.ds(ebase, EW + CH)], sidx)

        zg = z_hbm.at[pl.ds(0, CH)]

        def gslice(it):
            return g_hbm.at[sidx.at[pl.ds(it * CH, CH)]]

        def hslice(it):
            return h_hbm.at[sidx.at[pl.ds(it * CH, CH)]]

        def fetch_didx(it, buf):
            pltpu.async_copy(dst_hbm.at[pl.ds(ebase + it * CH, CH)], buf,
                             semd)

        def fetch_eav(it, buf):
            pltpu.async_copy(ea_hbm.at[pl.ds((ebase + it * CH) * 16, CH * 16)],
                             buf, seme)

        def drain_g(buf):
            pltpu.make_async_copy(zg, buf, semg).wait()

        def drain_gh():
            pltpu.make_async_copy(h_hbm.at[pl.ds(0, CH)], hrow, semg).wait()

        def drain_s(buf):
            pltpu.make_async_copy(zg, buf, sems).wait()

        def drain_d():
            pltpu.make_async_copy(dst_hbm.at[pl.ds(0, CH)], didx0, semd).wait()

        def drain_e():
            pltpu.make_async_copy(ea_hbm.at[pl.ds(0, CH * 16)], eav0,
                                  seme).wait()

        def zero_acc():
            pltpu.sync_copy(z_hbm, acc.at[pl.ds(s * RPT, RPT)])

        zero_acc()
        plsc.subcore_barrier()
        pltpu.async_copy(gslice(0), grow0, semg)
        fetch_didx(0, didx0)

        def gcn_pair(t, carry):
            it = 2 * t
            drain_g(grow0)

            @pl.when(t > 0)
            def _():
                drain_s(grow1)

            pltpu.async_copy(gslice(it + 1), grow1, semg)
            fetch_didx(it + 1, didx1)
            drain_d()
            pltpu.async_copy(grow0, acc.at[didx0], sems, add=True)
            drain_g(grow1)
            drain_s(grow0)
            pltpu.async_copy(gslice(it + 2), grow0, semg)
            fetch_didx(it + 2, didx0)
            drain_d()
            pltpu.async_copy(grow1, acc.at[didx1], sems, add=True)
            return carry

        lax.fori_loop(0, NIT // 2, gcn_pair, 0)
        drain_g(grow0)
        drain_d()
        drain_s(grow1)
        plsc.subcore_barrier()
        pltpu.sync_copy(acc.at[pl.ds(s * RPT, RPT)],
                        pg_hbm.at[c, pl.ds(s * RPT, RPT)])

        zero_acc()
        if O < 128:
            pltpu.sync_copy(zg, grow0)
            pltpu.sync_copy(zg, grow1)
        plsc.subcore_barrier()
        pltpu.async_copy(hslice(0), hrow, semg)
        fetch_didx(0, didx0)
        fetch_eav(0, eav0)

        def compute_msg(eav, gbuf):
            def edge(i, ecarry):
                ev = eav[pl.ds(i * 16, 16)]
                e0 = ev[0]
                e1 = ev[1]
                e2 = ev[2]
                e3 = ev[3]
                for j in range(O // 16):
                    v = (e0 * hrow[i, pl.ds(j * 16, 16)]
                         + e1 * hrow[i, pl.ds(O + j * 16, 16)]
                         + e2 * hrow[i, pl.ds(2 * O + j * 16, 16)]
                         + e3 * hrow[i, pl.ds(3 * O + j * 16, 16)])
                    gbuf[i, pl.ds(j * 16, 16)] = v
                return ecarry

            lax.fori_loop(0, CH, edge, 0)

        def ecc_pair(t, carry):
            it = 2 * t
            drain_gh()
            fetch_eav(it + 1, eav1)
            drain_e()
            compute_msg(eav0, grow0)

            @pl.when(t > 0)
            def _():
                drain_s(grow1)

            pltpu.async_copy(hslice(it + 1), hrow, semg)
            fetch_didx(it + 1, didx1)
            drain_d()
            pltpu.async_copy(grow0, acc.at[didx0], sems, add=True)
            drain_gh()
            fetch_eav(it + 2, eav0)
            drain_e()
            compute_msg(eav1, grow1)
            drain_s(grow0)
            pltpu.async_copy(hslice(it + 2), hrow, semg)
            fetch_didx(it + 2, didx0)
            drain_d()
            pltpu.async_copy(grow1, acc.at[didx1], sems, add=True)
            return carry

        lax.fori_loop(0, NIT // 2, ecc_pair, 0)
        drain_gh()
        drain_d()
        drain_e()
        drain_s(grow1)
        plsc.subcore_barrier()
        pltpu.sync_copy(acc.at[pl.ds(s * RPT, RPT)],
                        pe_hbm.at[c, pl.ds(s * RPT, RPT)])

    return k(gtab, htab, eafl, srcp, dstp, zrows)


def _full(shape):
    return pl.BlockSpec(shape, lambda i: tuple(0 for _ in shape))


def _tc1(degp, xp, w1, k1f, r1w):
    def body(degp_ref, x_ref, w1_ref, k1_ref, r1_ref,
             dis_ref, g1_ref, h1_ref, rr1_ref):
        d = degp_ref[0] + degp_ref[1]
        dis = lax.rsqrt(jnp.maximum(d[:, 0:1], 1.0))
        dis_ref[...] = dis
        xb = x_ref[...]
        g1_ref[...] = dis * jnp.dot(xb, w1_ref[...],
                                    preferred_element_type=jnp.float32)
        h1_ref[...] = jnp.dot(xb, k1_ref[...],
                              preferred_element_type=jnp.float32)
        rr1_ref[...] = jnp.dot(xb, r1_ref[...],
                               preferred_element_type=jnp.float32)

    return pl.pallas_call(
        body,
        grid=(GR,),
        in_specs=[
            pl.BlockSpec((NCORE, RB, 128), lambda i: (0, i, 0)),
            pl.BlockSpec((RB, F), lambda i: (i, 0)),
            _full((F, 128)),
            _full((F, DE * NN)),
            _full((F, NN)),
        ],
        out_specs=[
            pl.BlockSpec((RB, 1), lambda i: (i, 0)),
            pl.BlockSpec((RB, 128), lambda i: (i, 0)),
            pl.BlockSpec((RB, DE * NN), lambda i: (i, 0)),
            pl.BlockSpec((RB, NN), lambda i: (i, 0)),
        ],
        out_shape=[
            jax.ShapeDtypeStruct((NPAD, 1), jnp.float32),
            jax.ShapeDtypeStruct((NPAD, 128), jnp.float32),
            jax.ShapeDtypeStruct((NPAD, DE * NN), jnp.float32),
            jax.ShapeDtypeStruct((NPAD, NN), jnp.float32),
        ],
    )(degp, xp, w1, k1f, r1w)


def _tc2(p1c, rr1, dis, b1, be1, wb1, kb1f, rb1w):
    def body(pc_ref, rr1_ref, dis_ref, b1_ref, be1_ref,
             wb1_ref, kb1_ref, rb1_ref,
             o1_ref, o2_ref, g2_ref, h2_ref, rr2_ref):
        dis = dis_ref[...]
        o1 = jnp.maximum((pc_ref[0, :, :NN] + pc_ref[1, :, :NN]) * dis
                         + b1_ref[...], 0.0)
        o2 = jnp.maximum(pc_ref[0, :, NN:128] + pc_ref[1, :, NN:128]
                         + rr1_ref[...] + be1_ref[...], 0.0)
        o1_ref[...] = o1
        o2_ref[...] = o2
        g2_ref[...] = dis * jnp.dot(o1, wb1_ref[...],
                                    preferred_element_type=jnp.float32)
        h2_ref[...] = jnp.dot(o2, kb1_ref[...],
                              preferred_element_type=jnp.float32)
        rr2_ref[...] = jnp.dot(o2, rb1_ref[...],
                               preferred_element_type=jnp.float32)

    return pl.pallas_call(
        body,
        grid=(GR,),
        in_specs=[
            pl.BlockSpec((NCORE, RB, 128), lambda i: (0, i, 0)),
            pl.BlockSpec((RB, NN), lambda i: (i, 0)),
            pl.BlockSpec((RB, 1), lambda i: (i, 0)),
            _full((1, NN)),
            _full((1, NN)),
            _full((NN, 128)),
            _full((NN, DE * NN)),
            _full((NN, NN)),
        ],
        out_specs=[
            pl.BlockSpec((RB, NN), lambda i: (i, 0)),
            pl.BlockSpec((RB, NN), lambda i: (i, 0)),
            pl.BlockSpec((RB, 128), lambda i: (i, 0)),
            pl.BlockSpec((RB, DE * NN), lambda i: (i, 0)),
            pl.BlockSpec((RB, NN), lambda i: (i, 0)),
        ],
        out_shape=[
            jax.ShapeDtypeStruct((NPAD, NN), jnp.float32),
            jax.ShapeDtypeStruct((NPAD, NN), jnp.float32),
            jax.ShapeDtypeStruct((NPAD, 128), jnp.float32),
            jax.ShapeDtypeStruct((NPAD, DE * NN), jnp.float32),
            jax.ShapeDtypeStruct((NPAD, NN), jnp.float32),
        ],
    )(p1c, rr1, dis, b1, be1, wb1, kb1f, rb1w)


def _tc3(p2c, o1, o2, rr2, dis, bb1, beb1, wb2, kb2f, rb2w, proj1, proj2):
    NO = 2 * NN

    def body(pc_ref, o1_ref, o2_ref, rr2_ref, dis_ref,
             bb1_ref, beb1_ref, wb2_ref, kb2_ref, rb2_ref, pj1_ref, pj2_ref,
             g3_ref, h3_ref, rr3_ref, pr1_ref, pr2_ref):
        dis = dis_ref[...]
        o1b = jnp.maximum((pc_ref[0, :, :NN] + pc_ref[1, :, :NN]) * dis
                          + bb1_ref[...] + o1_ref[...], 0.0)
        o2b = jnp.maximum(pc_ref[0, :, NN:128] + pc_ref[1, :, NN:128]
                          + rr2_ref[...] + beb1_ref[...] + o2_ref[...], 0.0)
        g3_ref[...] = dis * jnp.dot(o1b, wb2_ref[...],
                                    preferred_element_type=jnp.float32)
        h3_ref[...] = jnp.dot(o2b, kb2_ref[...],
                              preferred_element_type=jnp.float32)
        rr3_ref[...] = jnp.dot(o2b, rb2_ref[...],
                               preferred_element_type=jnp.float32)
        pr1_ref[...] = jnp.dot(o1b, pj1_ref[...],
                               preferred_element_type=jnp.float32)
        pr2_ref[...] = jnp.dot(o2b, pj2_ref[...],
                               preferred_element_type=jnp.float32)

    return pl.pallas_call(
        body,
        grid=(GR,),
        in_specs=[
            pl.BlockSpec((NCORE, RB, 128), lambda i: (0, i, 0)),
            pl.BlockSpec((RB, NN), lambda i: (i, 0)),
            pl.BlockSpec((RB, NN), lambda i: (i, 0)),
            pl.BlockSpec((RB, NN), lambda i: (i, 0)),
            pl.BlockSpec((RB, 1), lambda i: (i, 0)),
            _full((1, NN)),
            _full((1, NN)),
            _full((NN, NO)),
            _full((NN, DE * NO)),
            _full((NN, NO)),
            _full((NN, NO)),
            _full((NN, NO)),
        ],
        out_specs=[
            pl.BlockSpec((RB, NO), lambda i: (i, 0)),
            pl.BlockSpec((RB, DE * NO), lambda i: (i, 0)),
            pl.BlockSpec((RB, NO), lambda i: (i, 0)),
            pl.BlockSpec((RB, NO), lambda i: (i, 0)),
            pl.BlockSpec((RB, NO), lambda i: (i, 0)),
        ],
        out_shape=[
            jax.ShapeDtypeStruct((NPAD, NO), jnp.float32),
            jax.ShapeDtypeStruct((NPAD, DE * NO), jnp.float32),
            jax.ShapeDtypeStruct((NPAD, NO), jnp.float32),
            jax.ShapeDtypeStruct((NPAD, NO), jnp.float32),
            jax.ShapeDtypeStruct((NPAD, NO), jnp.float32),
        ],
    )(p2c, o1, o2, rr2, dis, bb1, beb1, wb2, kb2f, rb2w, proj1, proj2)


def _tc4(p3g, p3e, rr3, pr1, pr2, dis, bb2, beb2, gids,
         d1w, d1b, d2w, d2b, d3wp, d3bp):
    NO = 2 * NN

    def body(pg_ref, pe_ref, rr3_ref, pr1_ref, pr2_ref, dis_ref,
             bb2_ref, beb2_ref, gid_ref, d1w_ref, d1b_ref, d2w_ref, d2b_ref,
             d3w_ref, d3b_ref, out_ref, acc_ref):
        i = pl.program_id(0)

        @pl.when(i == 0)
        def _():
            acc_ref[...] = jnp.zeros_like(acc_ref)

        dis = dis_ref[...]
        o1c = jnp.maximum((pg_ref[0] + pg_ref[1]) * dis + bb2_ref[...]
                          + pr1_ref[...], 0.0)
        o2c = jnp.maximum(pe_ref[0] + pe_ref[1] + rr3_ref[...] + beb2_ref[...]
                          + pr2_ref[...], 0.0)
        seg = (gid_ref[...] == lax.broadcasted_iota(jnp.int32, (RB, NG), 1)
               ).astype(jnp.float32)
        dn = (((0,), (0,)), ((), ()))
        acc_ref[:, 0:NO] += lax.dot_general(
            seg, o1c, dn, preferred_element_type=jnp.float32)
        acc_ref[:, NO:2 * NO] += lax.dot_general(
            seg, o2c, dn, preferred_element_type=jnp.float32)

        @pl.when(i == GR - 1)
        def _():
            p = acc_ref[...]
            h1 = jnp.maximum(
                jnp.dot(p, d1w_ref[...], preferred_element_type=jnp.float32)
                + d1b_ref[...], 0.0)
            h2 = jnp.maximum(
                jnp.dot(h1, d2w_ref[...], preferred_element_type=jnp.float32)
                + d2b_ref[...], 0.0)
            lg = jnp.dot(h2, d3w_ref[...],
                         preferred_element_type=jnp.float32) + d3b_ref[...]
            mx = jnp.max(lg, axis=1, keepdims=True)
            ex = jnp.exp(lg - mx)
            out_ref[...] = ex / jnp.sum(ex, axis=1, keepdims=True)

    return pl.pallas_call(
        body,
        grid=(GR,),
        in_specs=[
            pl.BlockSpec((NCORE, RB, NO), lambda i: (0, i, 0)),
            pl.BlockSpec((NCORE, RB, NO), lambda i: (0, i, 0)),
            pl.BlockSpec((RB, NO), lambda i: (i, 0)),
            pl.BlockSpec((RB, NO), lambda i: (i, 0)),
            pl.BlockSpec((RB, NO), lambda i: (i, 0)),
            pl.BlockSpec((RB, 1), lambda i: (i, 0)),
            _full((1, NO)),
            _full((1, NO)),
            pl.BlockSpec((RB, 1), lambda i: (i, 0)),
            _full((2 * NO, NN)),
            _full((1, NN)),
            _full((NN, NN // 2)),
            _full((1, NN // 2)),
            _full((NN // 2, 128)),
            _full((1, 128)),
        ],
        out_specs=pl.BlockSpec((NG, 128), lambda i: (0, 0)),
        out_shape=jax.ShapeDtypeStruct((NG, 128), jnp.float32),
        scratch_shapes=[pltpu.VMEM((NG, 2 * NO), jnp.float32)],
    )(p3g, p3e, rr3, pr1, pr2, dis, bb2, beb2, gids,
      d1w, d1b, d2w, d2b, d3wp, d3bp)


def kernel(x, edge_index, edge_attr, graph_ids,
           gcn1_W, gcn1_b, gcnb1_W, gcnb1_b, gcnb2_W, gcnb2_b, gcnb2_proj,
           ecc1_K, ecc1_root, ecc1_b, eccb1_K, eccb1_root, eccb1_b,
           eccb2_K, eccb2_root, eccb2_b, eccb2_proj,
           d1_W, d1_b, d2_W, d2_b, d3_W, d3_b):
    padE = EP - E
    srcp = jnp.concatenate(
        [edge_index[0], jnp.zeros((padE + 128,), jnp.int32)])
    dstp = jnp.concatenate(
        [edge_index[1], jnp.full((padE + 128,), N, jnp.int32)])
    eafl = jnp.pad(edge_attr, ((0, padE + 64), (0, 16 - DE))).reshape(-1)
    dstR128 = dstp[:EP].reshape(-1, 128)
    xp = jnp.pad(x, ((0, NPAD - N), (0, 0)))
    gidsp = jnp.concatenate(
        [graph_ids, jnp.full((NPAD - N,), NG, jnp.int32)]).reshape(NPAD, 1)

    k1f = jnp.transpose(ecc1_K, (1, 0, 2)).reshape(F, DE * NN)
    kb1f = jnp.transpose(eccb1_K, (1, 0, 2)).reshape(NN, DE * NN)
    kb2f = jnp.transpose(eccb2_K, (1, 0, 2)).reshape(NN, DE * 2 * NN)

    cones = jnp.concatenate(
        [jnp.ones((128, 1), jnp.float32), jnp.zeros((128, 127), jnp.float32)],
        axis=1)
    z128 = jnp.zeros((RPT, 128), jnp.float32)
    w1p = jnp.pad(gcn1_W, ((0, 0), (0, 128 - NN)))
    wb1p = jnp.pad(gcnb1_W, ((0, 0), (0, 128 - NN)))

    b1 = gcn1_b.reshape(1, NN)
    bb1 = gcnb1_b.reshape(1, NN)
    bb2 = gcnb2_b.reshape(1, 2 * NN)
    be1 = ecc1_b.reshape(1, NN)
    beb1 = eccb1_b.reshape(1, NN)
    beb2 = eccb2_b.reshape(1, 2 * NN)
    d1b = d1_b.reshape(1, NN)
    d2b = d2_b.reshape(1, NN // 2)
    d3wp = jnp.pad(d3_W, ((0, 0), (0, 128 - NOUT)))
    d3bp = jnp.concatenate(
        [d3_b, jnp.full((128 - NOUT,), -1e30, jnp.float32)]).reshape(1, 128)

    degp = _deg_pass(dstR128, cones, z128)
    dis, g1, h1, rr1 = _tc1(degp, xp, w1p, k1f, ecc1_root)
    p1c = _prop_fused(g1, h1, eafl, srcp, dstp, z128)
    o1, o2, g2, h2, rr2 = _tc2(p1c, rr1, dis, b1, be1,
                               wb1p, kb1f, eccb1_root)
    p2c = _prop_fused(g2, h2, eafl, srcp, dstp, z128)
    g3, h3, rr3, pr1, pr2 = _tc3(p2c, o1, o2, rr2, dis, bb1, beb1,
                                 gcnb2_W, kb2f, eccb2_root,
                                 gcnb2_proj, eccb2_proj)
    p3g, p3e = _prop_pass(g3, h3, eafl, srcp, dstp, z128, 2 * NN)
    outp = _tc4(p3g, p3e, rr3, pr1, pr2, dis, bb2, beb2, gidsp,
                d1_W, d1b, d2_W, d2b, d3wp, d3bp)
    return outp[:, :NOUT]


# speedup vs baseline: 3.1105x; 1.0012x over previous
import functools

import jax
import jax.numpy as jnp
from jax import lax
from jax.experimental import pallas as pl
from jax.experimental.pallas import tpu as pltpu
from jax.experimental.pallas import tpu_sc as plsc

N = 10000
E = 160000
F = 128
DE = 4
NN = 64
NG = 8
NOUT = 10

NPAD = 10112
NTILE = 16
NCORE = 2
NW = NCORE * NTILE
EP = 163840
EW = EP // NW
RPT = NPAD // NTILE
RB = 632
GR = NPAD // RB


def _sc_mesh():
    return plsc.VectorSubcoreMesh(core_axis_name="c", subcore_axis_name="s")


def _deg_pass(dstR, cones, zrows):
    DCH2 = 128
    NIT = EW // DCH2

    @functools.partial(
        pl.kernel,
        out_type=jax.ShapeDtypeStruct((NCORE, NPAD, 128), jnp.float32),
        mesh=_sc_mesh(),
        scratch_types=[
            pltpu.VMEM_SHARED((NPAD, 128), jnp.float32),
            pltpu.VMEM((NIT, DCH2), jnp.int32),
            pltpu.VMEM((DCH2, 128), jnp.float32),
            pltpu.SemaphoreType.DMA,
        ],
    )
    def k(dst_hbm, cones_hbm, z_hbm, out_hbm, acc, didx, ones_v, sems):
        c = lax.axis_index("c")
        s = lax.axis_index("s")
        wid = c * NTILE + s
        pltpu.sync_copy(cones_hbm, ones_v)
        pltpu.sync_copy(dst_hbm.at[pl.ds(wid * NIT, NIT)], didx)
        pltpu.sync_copy(z_hbm, acc.at[pl.ds(s * RPT, RPT)])
        plsc.subcore_barrier()

        def drain_s():
            pltpu.make_async_copy(z_hbm.at[pl.ds(0, DCH2)], ones_v, sems).wait()

        def body(it, carry):
            pltpu.async_copy(ones_v, acc.at[didx.at[it]], sems, add=True)

            @pl.when(it >= 2)
            def _():
                drain_s()

            return carry

        lax.fori_loop(0, NIT, body, 0)
        drain_s()
        drain_s()
        plsc.subcore_barrier()
        pltpu.sync_copy(acc.at[pl.ds(s * RPT, RPT)],
                        out_hbm.at[c, pl.ds(s * RPT, RPT)])

    return k(dstR, cones, zrows)


def _prop_fused(gtab, htab, eafl, srcp, dstp, zrows):
    O = NN
    CH = 64
    NIT = EW // CH

    @functools.partial(
        pl.kernel,
        out_type=jax.ShapeDtypeStruct((NCORE, NPAD, 128), jnp.float32),
        mesh=_sc_mesh(),
        scratch_types=[
            pltpu.VMEM_SHARED((NPAD, 128), jnp.float32),
            pltpu.VMEM((EW + CH,), jnp.int32),
            pltpu.VMEM((CH,), jnp.int32),
            pltpu.VMEM((CH,), jnp.int32),
            pltpu.VMEM((CH, 128), jnp.float32),
            pltpu.VMEM((CH, 128), jnp.float32),
            pltpu.VMEM((CH, DE * O), jnp.float32),
            pltpu.VMEM((CH * 16,), jnp.float32),
            pltpu.VMEM((CH * 16,), jnp.float32),
            pltpu.SemaphoreType.DMA,
            pltpu.SemaphoreType.DMA,
            pltpu.SemaphoreType.DMA,
            pltpu.SemaphoreType.DMA,
            pltpu.SemaphoreType.DMA,
        ],
    )
    def k(g_hbm, h_hbm, ea_hbm, src_hbm, dst_hbm, z_hbm, pc_hbm,
          acc, sidx, didx0, didx1, grow0, grow1, hrow, eav0, eav1,
          semg, semh, sems, semd, seme):
        c = lax.axis_index("c")
        s = lax.axis_index("s")
        wid = c * NTILE + s
        ebase = wid * EW
        pltpu.sync_copy(src_hbm.at[pl.ds(ebase, EW + CH)], sidx)

        zg = z_hbm.at[pl.ds(0, CH)]

        def gslice(it):
            return g_hbm.at[sidx.at[pl.ds(it * CH, CH)]]

        def hslice(it):
            return h_hbm.at[sidx.at[pl.ds(it * CH, CH)]]

        def fetch_didx(it, buf):
            pltpu.async_copy(dst_hbm.at[pl.ds(ebase + it * CH, CH)], buf,
                             semd)

        def fetch_eav(it, buf):
            pltpu.async_copy(ea_hbm.at[pl.ds((ebase + it * CH) * 16, CH * 16)],
                             buf, seme)

        def drain_g(buf):
            pltpu.make_async_copy(zg, buf, semg).wait()

        def drain_gh():
            pltpu.make_async_copy(h_hbm.at[pl.ds(0, CH)], hrow, semh).wait()

        def drain_s(buf):
            pltpu.make_async_copy(zg, buf, sems).wait()

        def drain_d():
            pltpu.make_async_copy(dst_hbm.at[pl.ds(0, CH)], didx0, semd).wait()

        def drain_e():
            pltpu.make_async_copy(ea_hbm.at[pl.ds(0, CH * 16)], eav0,
                                  seme).wait()

        def compute_msg(eav, gbuf):
            def edge(i2, ecarry):
                for u in range(2):
                    i = i2 * 2 + u
                    ev = eav[pl.ds(i * 16, 16)]
                    e0 = ev[0]
                    e1 = ev[1]
                    e2 = ev[2]
                    e3 = ev[3]
                    for j in range(O // 16):
                        v = (e0 * hrow[i, pl.ds(j * 16, 16)]
                             + e1 * hrow[i, pl.ds(O + j * 16, 16)]
                             + e2 * hrow[i, pl.ds(2 * O + j * 16, 16)]
                             + e3 * hrow[i, pl.ds(3 * O + j * 16, 16)])
                        gbuf[i, pl.ds(O + j * 16, 16)] = v
                return ecarry

            lax.fori_loop(0, CH // 2, edge, 0)

        pltpu.sync_copy(z_hbm, acc.at[pl.ds(s * RPT, RPT)])
        plsc.subcore_barrier()
        pltpu.async_copy(gslice(0), grow0, semg)
        pltpu.async_copy(hslice(0), hrow, semh)
        fetch_didx(0, didx0)
        fetch_eav(0, eav0)

        def pair(t, carry):
            it = 2 * t
            drain_g(grow0)
            drain_gh()
            fetch_eav(it + 1, eav1)
            drain_e()
            compute_msg(eav0, grow0)

            @pl.when(t > 0)
            def _():
                drain_s(grow1)

            pltpu.async_copy(gslice(it + 1), grow1, semg)
            pltpu.async_copy(hslice(it + 1), hrow, semh)
            fetch_didx(it + 1, didx1)
            drain_d()
            pltpu.async_copy(grow0, acc.at[didx0], sems, add=True)
            drain_g(grow1)
            drain_gh()
            fetch_eav(it + 2, eav0)
            drain_e()
            compute_msg(eav1, grow1)
            drain_s(grow0)
            pltpu.async_copy(gslice(it + 2), grow0, semg)
            pltpu.async_copy(hslice(it + 2), hrow, semh)
            fetch_didx(it + 2, didx0)
            drain_d()
            pltpu.async_copy(grow1, acc.at[didx1], sems, add=True)
            return carry

        lax.fori_loop(0, NIT // 2, pair, 0)
        drain_g(grow0)
        drain_gh()
        drain_d()
        drain_e()
        drain_s(grow1)
        plsc.subcore_barrier()
        pltpu.sync_copy(acc.at[pl.ds(s * RPT, RPT)],
                        pc_hbm.at[c, pl.ds(s * RPT, RPT)])

    return k(gtab, htab, eafl, srcp, dstp, zrows)


def _prop_pass(gtab, htab, eafl, srcp, dstp, zrows, O):
    HWID = DE * O
    CH = 64 if O <= 64 else 32
    NIT = EW // CH

    @functools.partial(
        pl.kernel,
        out_type=(jax.ShapeDtypeStruct((NCORE, NPAD, 128), jnp.float32),
                  jax.ShapeDtypeStruct((NCORE, NPAD, 128), jnp.float32)),
        mesh=_sc_mesh(),
        scratch_types=[
            pltpu.VMEM_SHARED((NPAD, 128), jnp.float32),
            pltpu.VMEM((EW + CH,), jnp.int32),
            pltpu.VMEM((CH,), jnp.int32),
            pltpu.VMEM((CH,), jnp.int32),
            pltpu.VMEM((CH, 128), jnp.float32),
            pltpu.VMEM((CH, 128), jnp.float32),
            pltpu.VMEM((CH, HWID), jnp.float32),
            pltpu.VMEM((CH * 16,), jnp.float32),
            pltpu.VMEM((CH * 16,), jnp.float32),
            pltpu.SemaphoreType.DMA,
            pltpu.SemaphoreType.DMA,
            pltpu.SemaphoreType.DMA,
            pltpu.SemaphoreType.DMA,
        ],
    )
    def k(g_hbm, h_hbm, ea_hbm, src_hbm, dst_hbm, z_hbm, pg_hbm, pe_hbm,
          acc, sidx, didx0, didx1, grow0, grow1, hrow, eav0, eav1,
          semg, sems, semd, seme):
        c = lax.axis_index("c")
        s = lax.axis_index("s")
        wid = c * NTILE + s
        ebase = wid * EW
        pltpu.sync_copy(src_hbm.at[pl.ds(ebase, EW + CH)], sidx)

        zg = z_hbm.at[pl.ds(0, CH)]

        def gslice(it):
            return g_hbm.at[sidx.at[pl.ds(it * CH, CH)]]

        def hslice(it):
            return h_hbm.at[sidx.at[pl.ds(it * CH, CH)]]

        def fetch_didx(it, buf):
            pltpu.async_copy(dst_hbm.at[pl.ds(ebase + it * CH, CH)], buf,
                             semd)

        def fetch_eav(it, buf):
            pltpu.async_copy(ea_hbm.at[pl.ds((ebase + it * CH) * 16, CH * 16)],
                             buf, seme)

        def drain_g(buf):
            pltpu.make_async_copy(zg, buf, semg).wait()

        def drain_gh():
            pltpu.make_async_copy(h_hbm.at[pl.ds(0, CH)], hrow, semg).wait()

        def drain_s(buf):
            pltpu.make_async_copy(zg, buf, sems).wait()

        def drain_d():
            pltpu.make_async_copy(dst_hbm.at[pl.ds(0, CH)], didx0, semd).wait()

        def drain_e():
            pltpu.make_async_copy(ea_hbm.at[pl.ds(0, CH * 16)], eav0,
                                  seme).wait()

        def zero_acc():
            pltpu.sync_copy(z_hbm, acc.at[pl.ds(s * RPT, RPT)])

        zero_acc()
        plsc.subcore_barrier()
        pltpu.async_copy(gslice(0), grow0, semg)
        fetch_didx(0, didx0)

        def gcn_pair(t, carry):
            it = 2 * t
            drain_g(grow0)

            @pl.when(t > 0)
            def _():
                drain_s(grow1)

            pltpu.async_copy(gslice(it + 1), grow1, semg)
            fetch_didx(it + 1, didx1)
            drain_d()
            pltpu.async_copy(grow0, acc.at[didx0], sems, add=True)
            drain_g(grow1)
            drain_s(grow0)
            pltpu.async_copy(gslice(it + 2), grow0, semg)
            fetch_didx(it + 2, didx0)
            drain_d()
            pltpu.async_copy(grow1, acc.at[didx1], sems, add=True)
            return carry

        lax.fori_loop(0, NIT // 2, gcn_pair, 0)
        drain_g(grow0)
        drain_d()
        drain_s(grow1)
        plsc.subcore_barrier()
        pltpu.sync_copy(acc.at[pl.ds(s * RPT, RPT)],
                        pg_hbm.at[c, pl.ds(s * RPT, RPT)])

        zero_acc()
        if O < 128:
            pltpu.sync_copy(zg, grow0)
            pltpu.sync_copy(zg, grow1)
        plsc.subcore_barrier()
        pltpu.async_copy(hslice(0), hrow, semg)
        fetch_didx(0, didx0)
        fetch_eav(0, eav0)

        def compute_msg(eav, gbuf):
            def edge(i2, ecarry):
                for u in range(2):
                    i = i2 * 2 + u
                    ev = eav[pl.ds(i * 16, 16)]
                    e0 = ev[0]
                    e1 = ev[1]
                    e2 = ev[2]
                    e3 = ev[3]
                    for j in range(O // 16):
                        v = (e0 * hrow[i, pl.ds(j * 16, 16)]
                             + e1 * hrow[i, pl.ds(O + j * 16, 16)]
                             + e2 * hrow[i, pl.ds(2 * O + j * 16, 16)]
                             + e3 * hrow[i, pl.ds(3 * O + j * 16, 16)])
                        gbuf[i, pl.ds(j * 16, 16)] = v
                return ecarry

            lax.fori_loop(0, CH // 2, edge, 0)

        def ecc_pair(t, carry):
            it = 2 * t
            drain_gh()
            fetch_eav(it + 1, eav1)
            drain_e()
            compute_msg(eav0, grow0)

            @pl.when(t > 0)
            def _():
                drain_s(grow1)

            pltpu.async_copy(hslice(it + 1), hrow, semg)
            fetch_didx(it + 1, didx1)
            drain_d()
            pltpu.async_copy(grow0, acc.at[didx0], sems, add=True)
            drain_gh()
            fetch_eav(it + 2, eav0)
            drain_e()
            compute_msg(eav1, grow1)
            drain_s(grow0)
            pltpu.async_copy(hslice(it + 2), hrow, semg)
            fetch_didx(it + 2, didx0)
            drain_d()
            pltpu.async_copy(grow1, acc.at[didx1], sems, add=True)
            return carry

        lax.fori_loop(0, NIT // 2, ecc_pair, 0)
        drain_gh()
        drain_d()
        drain_e()
        drain_s(grow1)
        plsc.subcore_barrier()
        pltpu.sync_copy(acc.at[pl.ds(s * RPT, RPT)],
                        pe_hbm.at[c, pl.ds(s * RPT, RPT)])

    return k(gtab, htab, eafl, srcp, dstp, zrows)


def _full(shape):
    return pl.BlockSpec(shape, lambda i: tuple(0 for _ in shape))


def _tc1(degp, xp, w1, k1f, r1w):
    def body(degp_ref, x_ref, w1_ref, k1_ref, r1_ref,
             dis_ref, g1_ref, h1_ref, rr1_ref):
        d = degp_ref[0] + degp_ref[1]
        dis = lax.rsqrt(jnp.maximum(d[:, 0:1], 1.0))
        dis_ref[...] = dis
        xb = x_ref[...]
        g1_ref[...] = dis * jnp.dot(xb, w1_ref[...],
                                    preferred_element_type=jnp.float32)
        h1_ref[...] = jnp.dot(xb, k1_ref[...],
                              preferred_element_type=jnp.float32)
        rr1_ref[...] = jnp.dot(xb, r1_ref[...],
                               preferred_element_type=jnp.float32)

    return pl.pallas_call(
        body,
        grid=(GR,),
        in_specs=[
            pl.BlockSpec((NCORE, RB, 128), lambda i: (0, i, 0)),
            pl.BlockSpec((RB, F), lambda i: (i, 0)),
            _full((F, 128)),
            _full((F, DE * NN)),
            _full((F, NN)),
        ],
        out_specs=[
            pl.BlockSpec((RB, 1), lambda i: (i, 0)),
            pl.BlockSpec((RB, 128), lambda i: (i, 0)),
            pl.BlockSpec((RB, DE * NN), lambda i: (i, 0)),
            pl.BlockSpec((RB, NN), lambda i: (i, 0)),
        ],
        out_shape=[
            jax.ShapeDtypeStruct((NPAD, 1), jnp.float32),
            jax.ShapeDtypeStruct((NPAD, 128), jnp.float32),
            jax.ShapeDtypeStruct((NPAD, DE * NN), jnp.float32),
            jax.ShapeDtypeStruct((NPAD, NN), jnp.float32),
        ],
    )(degp, xp, w1, k1f, r1w)


def _tc2(p1c, rr1, dis, b1, be1, wb1, kb1f, rb1w):
    def body(pc_ref, rr1_ref, dis_ref, b1_ref, be1_ref,
             wb1_ref, kb1_ref, rb1_ref,
             o1_ref, o2_ref, g2_ref, h2_ref, rr2_ref):
        dis = dis_ref[...]
        o1 = jnp.maximum((pc_ref[0, :, :NN] + pc_ref[1, :, :NN]) * dis
                         + b1_ref[...], 0.0)
        o2 = jnp.maximum(pc_ref[0, :, NN:128] + pc_ref[1, :, NN:128]
                         + rr1_ref[...] + be1_ref[...], 0.0)
        o1_ref[...] = o1
        o2_ref[...] = o2
        g2_ref[...] = dis * jnp.dot(o1, wb1_ref[...],
                                    preferred_element_type=jnp.float32)
        h2_ref[...] = jnp.dot(o2, kb1_ref[...],
                              preferred_element_type=jnp.float32)
        rr2_ref[...] = jnp.dot(o2, rb1_ref[...],
                               preferred_element_type=jnp.float32)

    return pl.pallas_call(
        body,
        grid=(GR,),
        in_specs=[
            pl.BlockSpec((NCORE, RB, 128), lambda i: (0, i, 0)),
            pl.BlockSpec((RB, NN), lambda i: (i, 0)),
            pl.BlockSpec((RB, 1), lambda i: (i, 0)),
            _full((1, NN)),
            _full((1, NN)),
            _full((NN, 128)),
            _full((NN, DE * NN)),
            _full((NN, NN)),
        ],
        out_specs=[
            pl.BlockSpec((RB, NN), lambda i: (i, 0)),
            pl.BlockSpec((RB, NN), lambda i: (i, 0)),
            pl.BlockSpec((RB, 128), lambda i: (i, 0)),
            pl.BlockSpec((RB, DE * NN), lambda i: (i, 0)),
            pl.BlockSpec((RB, NN), lambda i: (i, 0)),
        ],
        out_shape=[
            jax.ShapeDtypeStruct((NPAD, NN), jnp.float32),
            jax.ShapeDtypeStruct((NPAD, NN), jnp.float32),
            jax.ShapeDtypeStruct((NPAD, 128), jnp.float32),
            jax.ShapeDtypeStruct((NPAD, DE * NN), jnp.float32),
            jax.ShapeDtypeStruct((NPAD, NN), jnp.float32),
        ],
    )(p1c, rr1, dis, b1, be1, wb1, kb1f, rb1w)


def _tc3(p2c, o1, o2, rr2, dis, bb1, beb1, wb2, kb2f, rb2w, proj1, proj2):
    NO = 2 * NN

    def body(pc_ref, o1_ref, o2_ref, rr2_ref, dis_ref,
             bb1_ref, beb1_ref, wb2_ref, kb2_ref, rb2_ref, pj1_ref, pj2_ref,
             g3_ref, h3_ref, rr3_ref, pr1_ref, pr2_ref):
        dis = dis_ref[...]
        o1b = jnp.maximum((pc_ref[0, :, :NN] + pc_ref[1, :, :NN]) * dis
                          + bb1_ref[...] + o1_ref[...], 0.0)
        o2b = jnp.maximum(pc_ref[0, :, NN:128] + pc_ref[1, :, NN:128]
                          + rr2_ref[...] + beb1_ref[...] + o2_ref[...], 0.0)
        g3_ref[...] = dis * jnp.dot(o1b, wb2_ref[...],
                                    preferred_element_type=jnp.float32)
        h3_ref[...] = jnp.dot(o2b, kb2_ref[...],
                              preferred_element_type=jnp.float32)
        rr3_ref[...] = jnp.dot(o2b, rb2_ref[...],
                               preferred_element_type=jnp.float32)
        pr1_ref[...] = jnp.dot(o1b, pj1_ref[...],
                               preferred_element_type=jnp.float32)
        pr2_ref[...] = jnp.dot(o2b, pj2_ref[...],
                               preferred_element_type=jnp.float32)

    return pl.pallas_call(
        body,
        grid=(GR,),
        in_specs=[
            pl.BlockSpec((NCORE, RB, 128), lambda i: (0, i, 0)),
            pl.BlockSpec((RB, NN), lambda i: (i, 0)),
            pl.BlockSpec((RB, NN), lambda i: (i, 0)),
            pl.BlockSpec((RB, NN), lambda i: (i, 0)),
            pl.BlockSpec((RB, 1), lambda i: (i, 0)),
            _full((1, NN)),
            _full((1, NN)),
            _full((NN, NO)),
            _full((NN, DE * NO)),
            _full((NN, NO)),
            _full((NN, NO)),
            _full((NN, NO)),
        ],
        out_specs=[
            pl.BlockSpec((RB, NO), lambda i: (i, 0)),
            pl.BlockSpec((RB, DE * NO), lambda i: (i, 0)),
            pl.BlockSpec((RB, NO), lambda i: (i, 0)),
            pl.BlockSpec((RB, NO), lambda i: (i, 0)),
            pl.BlockSpec((RB, NO), lambda i: (i, 0)),
        ],
        out_shape=[
            jax.ShapeDtypeStruct((NPAD, NO), jnp.float32),
            jax.ShapeDtypeStruct((NPAD, DE * NO), jnp.float32),
            jax.ShapeDtypeStruct((NPAD, NO), jnp.float32),
            jax.ShapeDtypeStruct((NPAD, NO), jnp.float32),
            jax.ShapeDtypeStruct((NPAD, NO), jnp.float32),
        ],
    )(p2c, o1, o2, rr2, dis, bb1, beb1, wb2, kb2f, rb2w, proj1, proj2)


def _tc4(p3g, p3e, rr3, pr1, pr2, dis, bb2, beb2, gids,
         d1w, d1b, d2w, d2b, d3wp, d3bp):
    NO = 2 * NN

    def body(pg_ref, pe_ref, rr3_ref, pr1_ref, pr2_ref, dis_ref,
             bb2_ref, beb2_ref, gid_ref, d1w_ref, d1b_ref, d2w_ref, d2b_ref,
             d3w_ref, d3b_ref, out_ref, acc_ref):
        i = pl.program_id(0)

        @pl.when(i == 0)
        def _():
            acc_ref[...] = jnp.zeros_like(acc_ref)

        dis = dis_ref[...]
        o1c = jnp.maximum((pg_ref[0] + pg_ref[1]) * dis + bb2_ref[...]
                          + pr1_ref[...], 0.0)
        o2c = jnp.maximum(pe_ref[0] + pe_ref[1] + rr3_ref[...] + beb2_ref[...]
                          + pr2_ref[...], 0.0)
        seg = (gid_ref[...] == lax.broadcasted_iota(jnp.int32, (RB, NG), 1)
               ).astype(jnp.float32)
        dn = (((0,), (0,)), ((), ()))
        acc_ref[:, 0:NO] += lax.dot_general(
            seg, o1c, dn, preferred_element_type=jnp.float32)
        acc_ref[:, NO:2 * NO] += lax.dot_general(
            seg, o2c, dn, preferred_element_type=jnp.float32)

        @pl.when(i == GR - 1)
        def _():
            p = acc_ref[...]
            h1 = jnp.maximum(
                jnp.dot(p, d1w_ref[...], preferred_element_type=jnp.float32)
                + d1b_ref[...], 0.0)
            h2 = jnp.maximum(
                jnp.dot(h1, d2w_ref[...], preferred_element_type=jnp.float32)
                + d2b_ref[...], 0.0)
            lg = jnp.dot(h2, d3w_ref[...],
                         preferred_element_type=jnp.float32) + d3b_ref[...]
            mx = jnp.max(lg, axis=1, keepdims=True)
            ex = jnp.exp(lg - mx)
            out_ref[...] = ex / jnp.sum(ex, axis=1, keepdims=True)

    return pl.pallas_call(
        body,
        grid=(GR,),
        in_specs=[
            pl.BlockSpec((NCORE, RB, NO), lambda i: (0, i, 0)),
            pl.BlockSpec((NCORE, RB, NO), lambda i: (0, i, 0)),
            pl.BlockSpec((RB, NO), lambda i: (i, 0)),
            pl.BlockSpec((RB, NO), lambda i: (i, 0)),
            pl.BlockSpec((RB, NO), lambda i: (i, 0)),
            pl.BlockSpec((RB, 1), lambda i: (i, 0)),
            _full((1, NO)),
            _full((1, NO)),
            pl.BlockSpec((RB, 1), lambda i: (i, 0)),
            _full((2 * NO, NN)),
            _full((1, NN)),
            _full((NN, NN // 2)),
            _full((1, NN // 2)),
            _full((NN // 2, 128)),
            _full((1, 128)),
        ],
        out_specs=pl.BlockSpec((NG, 128), lambda i: (0, 0)),
        out_shape=jax.ShapeDtypeStruct((NG, 128), jnp.float32),
        scratch_shapes=[pltpu.VMEM((NG, 2 * NO), jnp.float32)],
    )(p3g, p3e, rr3, pr1, pr2, dis, bb2, beb2, gids,
      d1w, d1b, d2w, d2b, d3wp, d3bp)


def kernel(x, edge_index, edge_attr, graph_ids,
           gcn1_W, gcn1_b, gcnb1_W, gcnb1_b, gcnb2_W, gcnb2_b, gcnb2_proj,
           ecc1_K, ecc1_root, ecc1_b, eccb1_K, eccb1_root, eccb1_b,
           eccb2_K, eccb2_root, eccb2_b, eccb2_proj,
           d1_W, d1_b, d2_W, d2_b, d3_W, d3_b):
    padE = EP - E
    srcp = jnp.concatenate(
        [edge_index[0], jnp.zeros((padE + 128,), jnp.int32)])
    dstp = jnp.concatenate(
        [edge_index[1], jnp.full((padE + 128,), N, jnp.int32)])
    eafl = jnp.pad(edge_attr, ((0, padE + 64), (0, 16 - DE))).reshape(-1)
    dstR128 = dstp[:EP].reshape(-1, 128)
    xp = jnp.pad(x, ((0, NPAD - N), (0, 0)))
    gidsp = jnp.concatenate(
        [graph_ids, jnp.full((NPAD - N,), NG, jnp.int32)]).reshape(NPAD, 1)

    k1f = jnp.transpose(ecc1_K, (1, 0, 2)).reshape(F, DE * NN)
    kb1f = jnp.transpose(eccb1_K, (1, 0, 2)).reshape(NN, DE * NN)
    kb2f = jnp.transpose(eccb2_K, (1, 0, 2)).reshape(NN, DE * 2 * NN)

    cones = jnp.concatenate(
        [jnp.ones((128, 1), jnp.float32), jnp.zeros((128, 127), jnp.float32)],
        axis=1)
    z128 = jnp.zeros((RPT, 128), jnp.float32)
    w1p = jnp.pad(gcn1_W, ((0, 0), (0, 128 - NN)))
    wb1p = jnp.pad(gcnb1_W, ((0, 0), (0, 128 - NN)))

    b1 = gcn1_b.reshape(1, NN)
    bb1 = gcnb1_b.reshape(1, NN)
    bb2 = gcnb2_b.reshape(1, 2 * NN)
    be1 = ecc1_b.reshape(1, NN)
    beb1 = eccb1_b.reshape(1, NN)
    beb2 = eccb2_b.reshape(1, 2 * NN)
    d1b = d1_b.reshape(1, NN)
    d2b = d2_b.reshape(1, NN // 2)
    d3wp = jnp.pad(d3_W, ((0, 0), (0, 128 - NOUT)))
    d3bp = jnp.concatenate(
        [d3_b, jnp.full((128 - NOUT,), -1e30, jnp.float32)]).reshape(1, 128)

    degp = _deg_pass(dstR128, cones, z128)
    dis, g1, h1, rr1 = _tc1(degp, xp, w1p, k1f, ecc1_root)
    p1c = _prop_fused(g1, h1, eafl, srcp, dstp, z128)
    o1, o2, g2, h2, rr2 = _tc2(p1c, rr1, dis, b1, be1,
                               wb1p, kb1f, eccb1_root)
    p2c = _prop_fused(g2, h2, eafl, srcp, dstp, z128)
    g3, h3, rr3, pr1, pr2 = _tc3(p2c, o1, o2, rr2, dis, bb1, beb1,
                                 gcnb2_W, kb2f, eccb2_root,
                                 gcnb2_proj, eccb2_proj)
    p3g, p3e = _prop_pass(g3, h3, eafl, srcp, dstp, z128, 2 * NN)
    outp = _tc4(p3g, p3e, rr3, pr1, pr2, dis, bb2, beb2, gidsp,
                d1_W, d1b, d2_W, d2b, d3wp, d3bp)
    return outp[:, :NOUT]
